# baseline (speedup 1.0000x reference)
import functools

import jax
import jax.numpy as jnp
from jax import lax
from jax.experimental import pallas as pl
from jax.experimental.pallas import tpu as pltpu
from jax.experimental.pallas import tpu_sc as plsc

f32 = jnp.float32
i32 = jnp.int32

N = 10000
HEADS = 4
NC = 2
NS = 16
NW = NC * NS
CHUNK = 128
N_PAD = 10240
TRASH = N
RPT = N_PAD // NS
BLK = 1000


def _mesh():
    return plsc.VectorSubcoreMesh(
        core_axis_name="c", subcore_axis_name="s", num_cores=NC, num_subcores=NS
    )


@functools.lru_cache(maxsize=None)
def _make_deg(cpw):
    @functools.partial(
        pl.kernel,
        out_type=jax.ShapeDtypeStruct((NC, N_PAD), f32),
        mesh=_mesh(),
        scratch_types=[
            pltpu.VMEM_SHARED((N_PAD,), f32),
            pltpu.VMEM((CHUNK,), i32),
            pltpu.VMEM((CHUNK,), f32),
        ],
    )
    def deg_kernel(dst_hbm, zeros_hbm, out_hbm, acc_sh, dstidx, ones_v):
        c = lax.axis_index("c")
        s = lax.axis_index("s")
        w = c * NS + s
        for j in range(CHUNK // 16):
            ones_v[pl.ds(j * 16, 16)] = jnp.ones((16,), f32)
        pltpu.sync_copy(zeros_hbm, acc_sh.at[pl.ds(s * RPT, RPT)])
        plsc.subcore_barrier()

        def body(k, carry):
            base = (w * cpw + k) * CHUNK
            pltpu.sync_copy(dst_hbm.at[pl.ds(base, CHUNK)], dstidx)
            pltpu.sync_copy(ones_v, acc_sh.at[dstidx], add=True)
            return carry

        lax.fori_loop(0, cpw, body, 0)
        plsc.subcore_barrier()
        pltpu.sync_copy(
            acc_sh.at[pl.ds(s * RPT, RPT)], out_hbm.at[c, pl.ds(s * RPT, RPT)]
        )

    return deg_kernel


@functools.lru_cache(maxsize=None)
def _make_gcn2(cpw):
    @functools.partial(
        pl.kernel,
        out_type=jax.ShapeDtypeStruct((NC, N_PAD, 128), f32),
        mesh=_mesh(),
        scratch_types=[
            pltpu.VMEM_SHARED((N_PAD, 128), f32),
            pltpu.VMEM((2, CHUNK), i32),
            pltpu.VMEM((2, CHUNK), i32),
            pltpu.VMEM((CHUNK, 128), f32),
            pltpu.VMEM((CHUNK, 128), f32),
            pltpu.SemaphoreType.DMA,
            pltpu.SemaphoreType.DMA,
        ],
    )
    def gcn_kernel(pack_hbm, tab_hbm, zeros_hbm, out_hbm,
                   acc_sh, idxa, idxb, rowsa, rowsb, sema, semb):
        c = lax.axis_index("c")
        s = lax.axis_index("s")
        w = c * NS + s
        c0 = w * cpw
        pltpu.sync_copy(zeros_hbm, acc_sh.at[pl.ds(s * RPT, RPT)])
        plsc.subcore_barrier()
        pltpu.sync_copy(pack_hbm.at[pl.ds(2 * c0, 2)], idxa)
        pltpu.async_copy(tab_hbm.at[idxa.at[0]], rowsa, sema)

        def body(k2, carry):
            ck = c0 + 2 * k2
            pltpu.make_async_copy(tab_hbm.at[idxa.at[0]], rowsa, sema).wait()
            pltpu.sync_copy(pack_hbm.at[pl.ds(2 * (ck + 1), 2)], idxb)
            pltpu.async_copy(tab_hbm.at[idxb.at[0]], rowsb, semb)
            pltpu.sync_copy(rowsa, acc_sh.at[idxa.at[1]], add=True)
            pltpu.make_async_copy(tab_hbm.at[idxb.at[0]], rowsb, semb).wait()
            pltpu.sync_copy(pack_hbm.at[pl.ds(2 * (ck + 2), 2)], idxa)
            pltpu.async_copy(tab_hbm.at[idxa.at[0]], rowsa, sema)
            pltpu.sync_copy(rowsb, acc_sh.at[idxb.at[1]], add=True)
            return carry

        lax.fori_loop(0, cpw // 2, body, 0)
        pltpu.make_async_copy(tab_hbm.at[idxa.at[0]], rowsa, sema).wait()
        plsc.subcore_barrier()
        pltpu.sync_copy(
            acc_sh.at[pl.ds(s * RPT, RPT)], out_hbm.at[c, pl.ds(s * RPT, RPT)]
        )

    return gcn_kernel


@functools.lru_cache(maxsize=None)
def _make_gcn(d, cpw):
    @functools.partial(
        pl.kernel,
        out_type=jax.ShapeDtypeStruct((NC, N_PAD, d), f32),
        mesh=_mesh(),
        scratch_types=[
            pltpu.VMEM_SHARED((N_PAD, d), f32),
            pltpu.VMEM((CHUNK,), i32),
            pltpu.VMEM((CHUNK,), i32),
            pltpu.VMEM((CHUNK, d), f32),
            pltpu.SemaphoreType.DMA,
        ],
    )
    def gcn_kernel(src_hbm, dst_hbm, tab_hbm, zeros_hbm, out_hbm,
                   acc_sh, srcidx, dstidx, rows, sem):
        c = lax.axis_index("c")
        s = lax.axis_index("s")
        w = c * NS + s
        pltpu.sync_copy(zeros_hbm, acc_sh.at[pl.ds(s * RPT, RPT)])
        plsc.subcore_barrier()

        def body(k, carry):
            base = (w * cpw + k) * CHUNK
            pltpu.sync_copy(src_hbm.at[pl.ds(base, CHUNK)], srcidx)
            pltpu.sync_copy(dst_hbm.at[pl.ds(base, CHUNK)], dstidx)
            pltpu.async_copy(tab_hbm.at[srcidx], rows, sem).wait()
            pltpu.sync_copy(rows, acc_sh.at[dstidx], add=True)
            return carry

        lax.fori_loop(0, cpw, body, 0)
        plsc.subcore_barrier()
        pltpu.sync_copy(
            acc_sh.at[pl.ds(s * RPT, RPT)], out_hbm.at[c, pl.ds(s * RPT, RPT)]
        )

    return gcn_kernel


@functools.lru_cache(maxsize=None)
def _make_gat_a(epad, cpw):
    @functools.partial(
        pl.kernel,
        out_type=(
            jax.ShapeDtypeStruct((NC, HEADS * N_PAD), f32),
            jax.ShapeDtypeStruct((HEADS, epad), f32),
        ),
        mesh=_mesh(),
        scratch_types=[
            pltpu.VMEM_SHARED((HEADS * N_PAD,), f32),
            pltpu.VMEM((CHUNK,), i32),
            pltpu.VMEM((CHUNK,), i32),
            pltpu.VMEM((HEADS, CHUNK), i32),
            pltpu.VMEM((HEADS, CHUNK), i32),
            pltpu.VMEM((HEADS, CHUNK), f32),
            pltpu.VMEM((HEADS, CHUNK), f32),
            pltpu.VMEM((HEADS, CHUNK), f32),
            pltpu.VMEM((HEADS, CHUNK), f32),
            pltpu.SemaphoreType.DMA,
        ],
    )
    def gat_a_kernel(src_hbm, dst_hbm, at_hbm, dt_hbm, ct_hbm, zeros_hbm,
                     den_hbm, ex_hbm,
                     acc_sh, srcidx, dstidx, sidx, didx, av, dv, cv, exv, sem):
        c = lax.axis_index("c")
        s = lax.axis_index("s")
        w = c * NS + s
        pltpu.sync_copy(zeros_hbm, acc_sh.at[pl.ds(s * (HEADS * RPT), HEADS * RPT)])
        plsc.subcore_barrier()

        def body(k, carry):
            base = (w * cpw + k) * CHUNK
            pltpu.sync_copy(src_hbm.at[pl.ds(base, CHUNK)], srcidx)
            pltpu.sync_copy(dst_hbm.at[pl.ds(base, CHUNK)], dstidx)
            for h in range(HEADS):
                for j in range(CHUNK // 16):
                    sl = pl.ds(j * 16, 16)
                    sidx[h, sl] = srcidx[sl] + (h * N_PAD)
                    didx[h, sl] = dstidx[sl] + (h * N_PAD)
            for h in range(HEADS):
                pltpu.async_copy(at_hbm.at[sidx.at[h]], av.at[h], sem).wait()
                pltpu.async_copy(dt_hbm.at[didx.at[h]], dv.at[h], sem).wait()
                pltpu.async_copy(ct_hbm.at[didx.at[h]], cv.at[h], sem).wait()
            for h in range(HEADS):
                for j in range(CHUNK // 16):
                    sl = pl.ds(j * 16, 16)
                    x = av[h, sl] + dv[h, sl]
                    e = jnp.maximum(x, 0.0) + 0.2 * jnp.minimum(x, 0.0)
                    exv[h, sl] = jnp.exp(e - cv[h, sl])
            for h in range(HEADS):
                pltpu.sync_copy(exv.at[h], ex_hbm.at[h, pl.ds(base, CHUNK)])
                pltpu.sync_copy(exv.at[h], acc_sh.at[didx.at[h]], add=True)
            return carry

        lax.fori_loop(0, cpw, body, 0)
        plsc.subcore_barrier()
        pltpu.sync_copy(
            acc_sh.at[pl.ds(s * (HEADS * RPT), HEADS * RPT)],
            den_hbm.at[c, pl.ds(s * (HEADS * RPT), HEADS * RPT)],
        )

    return gat_a_kernel


@functools.lru_cache(maxsize=None)
def _make_gat_b(epad, cpw):
    @functools.partial(
        pl.kernel,
        out_type=jax.ShapeDtypeStruct((NC, N_PAD, 128), f32),
        mesh=_mesh(),
        scratch_types=[
            pltpu.VMEM_SHARED((N_PAD, 128), f32),
            pltpu.VMEM((CHUNK,), i32),
            pltpu.VMEM((CHUNK,), i32),
            pltpu.VMEM((HEADS, CHUNK), i32),
            pltpu.VMEM((HEADS, CHUNK), f32),
            pltpu.VMEM((HEADS, CHUNK), f32),
            pltpu.VMEM((HEADS * CHUNK,), f32),
            pltpu.VMEM((CHUNK // 2,), i32),
            pltpu.VMEM((CHUNK // 2, HEADS * 64), f32),
            pltpu.VMEM((CHUNK, 128), f32),
            pltpu.SemaphoreType.DMA,
            pltpu.SemaphoreType.DMA,
        ],
    )
    def gat_b_kernel(src_hbm, dst_hbm, ex_hbm, rdt_hbm, xwg_hbm, zeros_hbm,
                     out_hbm, acc_sh, srcidx, dstidx, didx, exv, rdv, alv,
                     srcidx_h, rows, outc, sem, sem_rows):
        c = lax.axis_index("c")
        s = lax.axis_index("s")
        w = c * NS + s
        pltpu.sync_copy(zeros_hbm, acc_sh.at[pl.ds(s * RPT, RPT)])

        def zrow(e, carry):
            for j in range(4):
                outc[e, pl.ds(64 + j * 16, 16)] = jnp.zeros((16,), f32)
            return carry

        lax.fori_loop(0, CHUNK, zrow, 0)
        plsc.subcore_barrier()

        def body(k, carry):
            base = (w * cpw + k) * CHUNK
            pltpu.sync_copy(dst_hbm.at[pl.ds(base, CHUNK)], dstidx)
            for h in range(HEADS):
                for j in range(CHUNK // 16):
                    sl = pl.ds(j * 16, 16)
                    didx[h, sl] = dstidx[sl] + (h * N_PAD)
            for h in range(HEADS):
                pltpu.sync_copy(ex_hbm.at[h, pl.ds(base, CHUNK)], exv.at[h])
                pltpu.async_copy(rdt_hbm.at[didx.at[h]], rdv.at[h], sem).wait()
            for h in range(HEADS):
                for j in range(CHUNK // 16):
                    sl = pl.ds(j * 16, 16)
                    alv[pl.ds(h * CHUNK + j * 16, 16)] = (
                        exv[h, sl] * rdv[h, sl] * 0.25
                    )
            for half in range(2):
                off = half * (CHUNK // 2)
                pltpu.sync_copy(src_hbm.at[pl.ds(base + off, CHUNK // 2)],
                                srcidx_h)
                pltpu.async_copy(
                    xwg_hbm.at[srcidx_h], rows, sem_rows,
                ).wait()

                def edge(e, carry2, off=off):
                    eo = e + off
                    e16 = (eo // 16) * 16
                    lanev = jnp.full((16,), eo - e16, dtype=i32)
                    acc = [jnp.zeros((16,), f32) for _ in range(4)]
                    for h in range(HEADS):
                        vv = alv[pl.ds(h * CHUNK + e16, 16)]
                        ah = lax.gather(
                            vv, lanev[:, None],
                            lax.GatherDimensionNumbers(
                                offset_dims=(), collapsed_slice_dims=(0,),
                                start_index_map=(0,)),
                            (1,),
                            mode=lax.GatherScatterMode.PROMISE_IN_BOUNDS,
                        )
                        for j in range(4):
                            acc[j] = acc[j] + ah * rows[e, pl.ds(h * 64 + j * 16, 16)]
                    for j in range(4):
                        outc[eo, pl.ds(j * 16, 16)] = acc[j]
                    return carry2

                lax.fori_loop(0, CHUNK // 2, edge, 0)
            pltpu.sync_copy(outc, acc_sh.at[dstidx], add=True)
            return carry

        lax.fori_loop(0, cpw, body, 0)
        plsc.subcore_barrier()
        pltpu.sync_copy(
            acc_sh.at[pl.ds(s * RPT, RPT)], out_hbm.at[c, pl.ds(s * RPT, RPT)]
        )

    return gat_b_kernel


def _row_spec(dd):
    return pl.BlockSpec((BLK, dd), lambda i: (i, 0))


def _rep_spec(r, dd):
    return pl.BlockSpec((r, dd), lambda i: (0, 0))


def _t0_body(x_ref, w_ref, d0_ref, d1_ref, dinv_ref, xs_ref):
    dinv = lax.rsqrt(d0_ref[...] + d1_ref[...])
    dinv_ref[...] = dinv
    xs_ref[...] = jnp.dot(x_ref[...], w_ref[...], preferred_element_type=f32) * dinv


def _t0(x, w0, d0, d1):
    return pl.pallas_call(
        _t0_body,
        grid=(N // BLK,),
        in_specs=[_row_spec(128), _rep_spec(128, 128), _row_spec(1), _row_spec(1)],
        out_specs=[_row_spec(1), _row_spec(128)],
        out_shape=[
            jax.ShapeDtypeStruct((N, 1), f32),
            jax.ShapeDtypeStruct((N, 128), f32),
        ],
    )(x, w0, d0, d1)


def _t1_body(p0, p1, dinv, b0, w1, h1_ref, xs_ref):
    dv = dinv[...]
    h1 = jnp.maximum((p0[...] + p1[...]) * dv + b0[...], 0.0)
    h1_ref[...] = h1
    xs_ref[...] = jnp.dot(h1, w1[...], preferred_element_type=f32) * dv


def _t1(p0, p1, dinv, b0, w1):
    return pl.pallas_call(
        _t1_body,
        grid=(N // BLK,),
        in_specs=[_row_spec(128), _row_spec(128), _row_spec(1),
                  _rep_spec(1, 128), _rep_spec(128, 128)],
        out_specs=[_row_spec(128), _row_spec(128)],
        out_shape=[
            jax.ShapeDtypeStruct((N, 128), f32),
            jax.ShapeDtypeStruct((N, 128), f32),
        ],
    )(p0, p1, dinv, b0, w1)


def _t2_body(p0, p1, dinv, b1, h1, w2, xs_ref):
    dv = dinv[...]
    h2 = h1[...] + jnp.maximum((p0[...] + p1[...]) * dv + b1[...], 0.0)
    xs_ref[...] = jnp.dot(h2, w2[...], preferred_element_type=f32) * dv


def _t2(p0, p1, dinv, b1, h1, w2):
    return pl.pallas_call(
        _t2_body,
        grid=(N // BLK,),
        in_specs=[_row_spec(128), _row_spec(128), _row_spec(1),
                  _rep_spec(1, 128), _row_spec(128), _rep_spec(128, 64)],
        out_specs=[_row_spec(64)],
        out_shape=[jax.ShapeDtypeStruct((N, 64), f32)],
    )(p0, p1, dinv, b1, h1, w2)[0]


def _t3_body(p0, p1, dinv, b2, wg, acomb, xwg_ref, aout_ref):
    dv = dinv[...]
    h3 = jnp.maximum((p0[...] + p1[...]) * dv + b2[...], 0.0)
    xwg = jnp.dot(h3, wg[...], preferred_element_type=f32)
    xwg_ref[...] = xwg
    aout_ref[...] = jnp.dot(xwg, acomb[...], preferred_element_type=f32)


def _t3(p0, p1, dinv, b2, wg, acomb):
    return pl.pallas_call(
        _t3_body,
        grid=(N // BLK,),
        in_specs=[_row_spec(64), _row_spec(64), _row_spec(1),
                  _rep_spec(1, 64), _rep_spec(64, 256), _rep_spec(256, 128)],
        out_specs=[_row_spec(256), _row_spec(128)],
        out_shape=[
            jax.ShapeDtypeStruct((N, 256), f32),
            jax.ShapeDtypeStruct((N, 128), f32),
        ],
    )(p0, p1, dinv, b2, wg, acomb)


def _t3b_body(asrc_ref, adst_ref, c_ref):
    amax = jnp.max(asrc_ref[...], axis=0, keepdims=True)
    x = amax + adst_ref[...]
    c_ref[...] = jnp.maximum(x, 0.0) + 0.2 * jnp.minimum(x, 0.0)


def _t3b(asrc, adst):
    return pl.pallas_call(
        _t3b_body,
        grid=(1,),
        in_specs=[_rep_spec(N, HEADS), _rep_spec(N, HEADS)],
        out_specs=[_rep_spec(N, HEADS)],
        out_shape=[jax.ShapeDtypeStruct((N, HEADS), f32)],
    )(asrc, adst)[0]


def _t4_body(d0_ref, d1_ref, r_ref):
    r_ref[...] = 1.0 / (d0_ref[...] + d1_ref[...] + 1e-16)


def _t4(d0, d1):
    r = HEADS * N_PAD // 128
    return pl.pallas_call(
        _t4_body,
        grid=(1,),
        in_specs=[_rep_spec(r, 128), _rep_spec(r, 128)],
        out_specs=[_rep_spec(r, 128)],
        out_shape=[jax.ShapeDtypeStruct((r, 128), f32)],
    )(d0, d1)[0]


def _t5_body(p0, p1, bg, wcomb, bcomb, h4_ref, ge_ref, bo_ref):
    h4 = p0[...] + p1[...] + bg[...]
    h4_ref[...] = h4

    @pl.when(pl.program_id(0) == 0)
    def _():
        ge_ref[...] = jnp.zeros_like(ge_ref)

    ge_ref[...] += jnp.sum(h4, axis=0, keepdims=True) * (1.0 / N)
    bo_ref[...] = jnp.dot(h4, wcomb[...], preferred_element_type=f32) + bcomb[...]


def _t5(p0, p1, bg, wcomb, bcomb):
    return pl.pallas_call(
        _t5_body,
        grid=(N // BLK,),
        in_specs=[_row_spec(64), _row_spec(64), _rep_spec(1, 64),
                  _rep_spec(64, 128), _rep_spec(1, 128)],
        out_specs=[_row_spec(64), _rep_spec(1, 64), _row_spec(128)],
        out_shape=[
            jax.ShapeDtypeStruct((N, 64), f32),
            jax.ShapeDtypeStruct((1, 64), f32),
            jax.ShapeDtypeStruct((N, 128), f32),
        ],
    )(p0, p1, bg, wcomb, bcomb)


def _pad_rows(t):
    return jnp.pad(t, ((0, N_PAD - N), (0, 0)))


def _flat_t(a):
    return jnp.pad(a.T, ((0, 0), (0, N_PAD - N))).reshape(-1)


def kernel(x, edge_index, W0, b0, W1, b1, W2, b2, Wg, att_src, att_dst, bg,
           Wb, bb, Wo, bo):
    e_in = edge_index.shape[1]
    etot = e_in + N
    cpw = -(-etot // (NW * CHUNK))
    cpw += cpw % 2
    epad = cpw * NW * CHUNK

    loop = jnp.arange(N, dtype=i32)
    fill = jnp.full((epad - etot,), TRASH, dtype=i32)
    srcp = jnp.concatenate([edge_index[0], loop, fill])
    dstp = jnp.concatenate([edge_index[1], loop, fill])
    nch = epad // CHUNK
    pack = jnp.stack([srcp.reshape(nch, CHUNK), dstp.reshape(nch, CHUNK)],
                     axis=1).reshape(2 * nch, CHUNK)
    pack = jnp.concatenate([pack, jnp.zeros((2, CHUNK), i32)], axis=0)

    z1 = jnp.zeros((RPT,), f32)
    z128 = jnp.zeros((RPT, 128), f32)
    z64 = jnp.zeros((RPT, 64), f32)
    z4 = jnp.zeros((HEADS * RPT,), f32)

    deg = _make_deg(cpw)(dstp, z1)
    d0 = deg[0, :N].reshape(N, 1)
    d1 = deg[1, :N].reshape(N, 1)
    dinv, xw0s = _t0(x, W0, d0, d1)

    gcn128 = _make_gcn2(cpw)
    p = gcn128(pack, _pad_rows(xw0s), z128)
    h1, xw1s = _t1(p[0, :N], p[1, :N], dinv, b0.reshape(1, -1), W1)

    p = gcn128(pack, _pad_rows(xw1s), z128)
    xw2s = _t2(p[0, :N], p[1, :N], dinv, b1.reshape(1, -1), h1, W2)

    xw2s_wide = jnp.pad(xw2s, ((0, 0), (0, 64)))
    p = gcn128(pack, _pad_rows(xw2s_wide), z128)
    p = p[:, :, :64]

    acomb = jnp.zeros((256, 128), f32)
    for h in range(HEADS):
        acomb = acomb.at[h * 64:(h + 1) * 64, h].set(att_src[h])
        acomb = acomb.at[h * 64:(h + 1) * 64, HEADS + h].set(att_dst[h])
    xwg, aout = _t3(p[0, :N], p[1, :N], dinv, b2.reshape(1, -1), Wg, acomb)
    asrc = aout[:, :HEADS]
    adst = aout[:, HEADS:2 * HEADS]
    cvals = _t3b(asrc, adst)

    den, ex = _make_gat_a(epad, cpw)(
        srcp, dstp, _flat_t(asrc), _flat_t(adst), _flat_t(cvals), z4
    )
    r = HEADS * N_PAD // 128
    rden = _t4(den[0].reshape(r, 128), den[1].reshape(r, 128)).reshape(-1)

    pg = _make_gat_b(epad, cpw)(srcp, dstp, ex, rden, _pad_rows(xwg), z128)

    wcomb = jnp.zeros((64, 128), f32)
    wcomb = wcomb.at[:, :2].set(Wb)
    wcomb = wcomb.at[:, 2:8].set(Wo)
    bcomb = jnp.zeros((1, 128), f32)
    bcomb = bcomb.at[0, :2].set(bb)
    bcomb = bcomb.at[0, 2:8].set(bo)

    h4, ge, bo_full = _t5(pg[0, :N, :64], pg[1, :N, :64], bg.reshape(1, -1), wcomb, bcomb)
    return h4, ge, bo_full[:, :2], bo_full[:, 2:8]


# speedup vs baseline: 1.3633x; 1.3633x over previous
import functools

import jax
import jax.numpy as jnp
from jax import lax
from jax.experimental import pallas as pl
from jax.experimental.pallas import tpu as pltpu
from jax.experimental.pallas import tpu_sc as plsc

f32 = jnp.float32
i32 = jnp.int32

N = 10000
HEADS = 4
NC = 2
NS = 16
NW = NC * NS
CHUNK = 128
N_PAD = 10240
TRASH = N
RPT = N_PAD // NS
BLK = 1000


def _mesh():
    return plsc.VectorSubcoreMesh(
        core_axis_name="c", subcore_axis_name="s", num_cores=NC, num_subcores=NS
    )


@functools.lru_cache(maxsize=None)
def _make_deg(cpw):
    @functools.partial(
        pl.kernel,
        out_type=jax.ShapeDtypeStruct((NC, N_PAD), f32),
        mesh=_mesh(),
        scratch_types=[
            pltpu.VMEM_SHARED((N_PAD,), f32),
            pltpu.VMEM((CHUNK,), i32),
            pltpu.VMEM((CHUNK,), f32),
        ],
    )
    def deg_kernel(dst_hbm, zeros_hbm, out_hbm, acc_sh, dstidx, ones_v):
        c = lax.axis_index("c")
        s = lax.axis_index("s")
        w = c * NS + s
        for j in range(CHUNK // 16):
            ones_v[pl.ds(j * 16, 16)] = jnp.ones((16,), f32)
        pltpu.sync_copy(zeros_hbm, acc_sh.at[pl.ds(s * RPT, RPT)])
        plsc.subcore_barrier()

        def body(k, carry):
            base = (w * cpw + k) * CHUNK
            pltpu.sync_copy(dst_hbm.at[pl.ds(base, CHUNK)], dstidx)
            pltpu.sync_copy(ones_v, acc_sh.at[dstidx], add=True)
            return carry

        lax.fori_loop(0, cpw, body, 0)
        plsc.subcore_barrier()
        pltpu.sync_copy(
            acc_sh.at[pl.ds(s * RPT, RPT)], out_hbm.at[c, pl.ds(s * RPT, RPT)]
        )

    return deg_kernel


@functools.lru_cache(maxsize=None)
def _make_gcn2(cpw):
    @functools.partial(
        pl.kernel,
        out_type=jax.ShapeDtypeStruct((NC, N_PAD, 128), f32),
        mesh=_mesh(),
        scratch_types=[
            pltpu.VMEM_SHARED((N_PAD, 128), f32),
            pltpu.VMEM((2, CHUNK), i32),
            pltpu.VMEM((2, CHUNK), i32),
            pltpu.VMEM((CHUNK, 128), f32),
            pltpu.VMEM((CHUNK, 128), f32),
            pltpu.SemaphoreType.DMA,
            pltpu.SemaphoreType.DMA,
        ],
    )
    def gcn_kernel(pack_hbm, tab_hbm, zeros_hbm, out_hbm,
                   acc_sh, idxa, idxb, rowsa, rowsb, sema, semb):
        c = lax.axis_index("c")
        s = lax.axis_index("s")
        w = c * NS + s
        c0 = w * cpw
        pltpu.sync_copy(zeros_hbm, acc_sh.at[pl.ds(s * RPT, RPT)])
        plsc.subcore_barrier()
        pltpu.sync_copy(pack_hbm.at[pl.ds(2 * c0, 2)], idxa)
        pltpu.async_copy(tab_hbm.at[idxa.at[0]], rowsa, sema)

        def body(k2, carry):
            ck = c0 + 2 * k2
            pltpu.make_async_copy(tab_hbm.at[idxa.at[0]], rowsa, sema).wait()
            pltpu.sync_copy(pack_hbm.at[pl.ds(2 * (ck + 1), 2)], idxb)
            pltpu.async_copy(tab_hbm.at[idxb.at[0]], rowsb, semb)
            pltpu.sync_copy(rowsa, acc_sh.at[idxa.at[1]], add=True)
            pltpu.make_async_copy(tab_hbm.at[idxb.at[0]], rowsb, semb).wait()
            pltpu.sync_copy(pack_hbm.at[pl.ds(2 * (ck + 2), 2)], idxa)
            pltpu.async_copy(tab_hbm.at[idxa.at[0]], rowsa, sema)
            pltpu.sync_copy(rowsb, acc_sh.at[idxb.at[1]], add=True)
            return carry

        lax.fori_loop(0, cpw // 2, body, 0)
        pltpu.make_async_copy(tab_hbm.at[idxa.at[0]], rowsa, sema).wait()
        plsc.subcore_barrier()
        pltpu.sync_copy(
            acc_sh.at[pl.ds(s * RPT, RPT)], out_hbm.at[c, pl.ds(s * RPT, RPT)]
        )

    return gcn_kernel


@functools.lru_cache(maxsize=None)
def _make_gcn(d, cpw):
    @functools.partial(
        pl.kernel,
        out_type=jax.ShapeDtypeStruct((NC, N_PAD, d), f32),
        mesh=_mesh(),
        scratch_types=[
            pltpu.VMEM_SHARED((N_PAD, d), f32),
            pltpu.VMEM((CHUNK,), i32),
            pltpu.VMEM((CHUNK,), i32),
            pltpu.VMEM((CHUNK, d), f32),
            pltpu.SemaphoreType.DMA,
        ],
    )
    def gcn_kernel(src_hbm, dst_hbm, tab_hbm, zeros_hbm, out_hbm,
                   acc_sh, srcidx, dstidx, rows, sem):
        c = lax.axis_index("c")
        s = lax.axis_index("s")
        w = c * NS + s
        pltpu.sync_copy(zeros_hbm, acc_sh.at[pl.ds(s * RPT, RPT)])
        plsc.subcore_barrier()

        def body(k, carry):
            base = (w * cpw + k) * CHUNK
            pltpu.sync_copy(src_hbm.at[pl.ds(base, CHUNK)], srcidx)
            pltpu.sync_copy(dst_hbm.at[pl.ds(base, CHUNK)], dstidx)
            pltpu.async_copy(tab_hbm.at[srcidx], rows, sem).wait()
            pltpu.sync_copy(rows, acc_sh.at[dstidx], add=True)
            return carry

        lax.fori_loop(0, cpw, body, 0)
        plsc.subcore_barrier()
        pltpu.sync_copy(
            acc_sh.at[pl.ds(s * RPT, RPT)], out_hbm.at[c, pl.ds(s * RPT, RPT)]
        )

    return gcn_kernel


@functools.lru_cache(maxsize=None)
def _make_gat_a(epad, cpw):
    @functools.partial(
        pl.kernel,
        out_type=(
            jax.ShapeDtypeStruct((NC, HEADS * N_PAD), f32),
            jax.ShapeDtypeStruct((HEADS, epad), f32),
        ),
        mesh=_mesh(),
        scratch_types=[
            pltpu.VMEM_SHARED((HEADS * N_PAD,), f32),
            pltpu.VMEM((CHUNK,), i32),
            pltpu.VMEM((CHUNK,), i32),
            pltpu.VMEM((HEADS, CHUNK), i32),
            pltpu.VMEM((HEADS, CHUNK), i32),
            pltpu.VMEM((HEADS, CHUNK), f32),
            pltpu.VMEM((HEADS, CHUNK), f32),
            pltpu.VMEM((HEADS, CHUNK), f32),
            pltpu.VMEM((HEADS, CHUNK), f32),
            pltpu.SemaphoreType.DMA,
        ],
    )
    def gat_a_kernel(src_hbm, dst_hbm, at_hbm, dt_hbm, ct_hbm, zeros_hbm,
                     den_hbm, ex_hbm,
                     acc_sh, srcidx, dstidx, sidx, didx, av, dv, cv, exv, sem):
        c = lax.axis_index("c")
        s = lax.axis_index("s")
        w = c * NS + s
        pltpu.sync_copy(zeros_hbm, acc_sh.at[pl.ds(s * (HEADS * RPT), HEADS * RPT)])
        plsc.subcore_barrier()

        def body(k, carry):
            base = (w * cpw + k) * CHUNK
            pltpu.sync_copy(src_hbm.at[pl.ds(base, CHUNK)], srcidx)
            pltpu.sync_copy(dst_hbm.at[pl.ds(base, CHUNK)], dstidx)
            for h in range(HEADS):
                for j in range(CHUNK // 16):
                    sl = pl.ds(j * 16, 16)
                    sidx[h, sl] = srcidx[sl] + (h * N_PAD)
                    didx[h, sl] = dstidx[sl] + (h * N_PAD)
            for h in range(HEADS):
                pltpu.async_copy(at_hbm.at[sidx.at[h]], av.at[h], sem).wait()
                pltpu.async_copy(dt_hbm.at[didx.at[h]], dv.at[h], sem).wait()
                pltpu.async_copy(ct_hbm.at[didx.at[h]], cv.at[h], sem).wait()
            for h in range(HEADS):
                for j in range(CHUNK // 16):
                    sl = pl.ds(j * 16, 16)
                    x = av[h, sl] + dv[h, sl]
                    e = jnp.maximum(x, 0.0) + 0.2 * jnp.minimum(x, 0.0)
                    exv[h, sl] = jnp.exp(e - cv[h, sl])
            for h in range(HEADS):
                pltpu.sync_copy(exv.at[h], ex_hbm.at[h, pl.ds(base, CHUNK)])
                pltpu.sync_copy(exv.at[h], acc_sh.at[didx.at[h]], add=True)
            return carry

        lax.fori_loop(0, cpw, body, 0)
        plsc.subcore_barrier()
        pltpu.sync_copy(
            acc_sh.at[pl.ds(s * (HEADS * RPT), HEADS * RPT)],
            den_hbm.at[c, pl.ds(s * (HEADS * RPT), HEADS * RPT)],
        )

    return gat_a_kernel


@functools.lru_cache(maxsize=None)
def _make_gat_b(epad, cpw):
    @functools.partial(
        pl.kernel,
        out_type=jax.ShapeDtypeStruct((NC, N_PAD, 128), f32),
        mesh=_mesh(),
        scratch_types=[
            pltpu.VMEM_SHARED((N_PAD, 128), f32),
            pltpu.VMEM((CHUNK,), i32),
            pltpu.VMEM((CHUNK,), i32),
            pltpu.VMEM((HEADS, CHUNK), i32),
            pltpu.VMEM((HEADS, CHUNK), f32),
            pltpu.VMEM((HEADS, CHUNK), f32),
            pltpu.VMEM((HEADS * CHUNK,), f32),
            pltpu.VMEM((CHUNK // 2,), i32),
            pltpu.VMEM((CHUNK // 2, HEADS * 64), f32),
            pltpu.VMEM((CHUNK, 128), f32),
            pltpu.SemaphoreType.DMA,
            pltpu.SemaphoreType.DMA,
        ],
    )
    def gat_b_kernel(src_hbm, dst_hbm, ex_hbm, rdt_hbm, xwg_hbm, zeros_hbm,
                     out_hbm, acc_sh, srcidx, dstidx, didx, exv, rdv, alv,
                     srcidx_h, rows, outc, sem, sem_rows):
        c = lax.axis_index("c")
        s = lax.axis_index("s")
        w = c * NS + s
        pltpu.sync_copy(zeros_hbm, acc_sh.at[pl.ds(s * RPT, RPT)])

        def zrow(e, carry):
            for j in range(4):
                outc[e, pl.ds(64 + j * 16, 16)] = jnp.zeros((16,), f32)
            return carry

        lax.fori_loop(0, CHUNK, zrow, 0)
        plsc.subcore_barrier()

        def body(k, carry):
            base = (w * cpw + k) * CHUNK
            pltpu.sync_copy(dst_hbm.at[pl.ds(base, CHUNK)], dstidx)
            for h in range(HEADS):
                for j in range(CHUNK // 16):
                    sl = pl.ds(j * 16, 16)
                    didx[h, sl] = dstidx[sl] + (h * N_PAD)
            for h in range(HEADS):
                pltpu.sync_copy(ex_hbm.at[h, pl.ds(base, CHUNK)], exv.at[h])
                pltpu.async_copy(rdt_hbm.at[didx.at[h]], rdv.at[h], sem).wait()
            for h in range(HEADS):
                for j in range(CHUNK // 16):
                    sl = pl.ds(j * 16, 16)
                    alv[pl.ds(h * CHUNK + j * 16, 16)] = (
                        exv[h, sl] * rdv[h, sl] * 0.25
                    )
            for half in range(2):
                off = half * (CHUNK // 2)
                pltpu.sync_copy(src_hbm.at[pl.ds(base + off, CHUNK // 2)],
                                srcidx_h)
                pltpu.async_copy(
                    xwg_hbm.at[srcidx_h], rows, sem_rows,
                ).wait()

                def edge(e, carry2, off=off):
                    eo = e + off
                    e16 = (eo // 16) * 16
                    lanev = jnp.full((16,), eo - e16, dtype=i32)
                    acc = [jnp.zeros((16,), f32) for _ in range(4)]
                    for h in range(HEADS):
                        vv = alv[pl.ds(h * CHUNK + e16, 16)]
                        ah = lax.gather(
                            vv, lanev[:, None],
                            lax.GatherDimensionNumbers(
                                offset_dims=(), collapsed_slice_dims=(0,),
                                start_index_map=(0,)),
                            (1,),
                            mode=lax.GatherScatterMode.PROMISE_IN_BOUNDS,
                        )
                        for j in range(4):
                            acc[j] = acc[j] + ah * rows[e, pl.ds(h * 64 + j * 16, 16)]
                    for j in range(4):
                        outc[eo, pl.ds(j * 16, 16)] = acc[j]
                    return carry2

                lax.fori_loop(0, CHUNK // 2, edge, 0)
            pltpu.sync_copy(outc, acc_sh.at[dstidx], add=True)
            return carry

        lax.fori_loop(0, cpw, body, 0)
        plsc.subcore_barrier()
        pltpu.sync_copy(
            acc_sh.at[pl.ds(s * RPT, RPT)], out_hbm.at[c, pl.ds(s * RPT, RPT)]
        )

    return gat_b_kernel


def _row_spec(dd):
    return pl.BlockSpec((BLK, dd), lambda i: (i, 0))


def _rep_spec(r, dd):
    return pl.BlockSpec((r, dd), lambda i: (0, 0))


def _t0_body(x_ref, w_ref, d0_ref, d1_ref, dinv_ref, xs_ref):
    dinv = lax.rsqrt(d0_ref[...] + d1_ref[...])
    dinv_ref[...] = dinv
    xs_ref[...] = jnp.dot(x_ref[...], w_ref[...], preferred_element_type=f32) * dinv


def _t0(x, w0, d0, d1):
    return pl.pallas_call(
        _t0_body,
        grid=(N // BLK,),
        in_specs=[_row_spec(128), _rep_spec(128, 128), _row_spec(1), _row_spec(1)],
        out_specs=[_row_spec(1), _row_spec(128)],
        out_shape=[
            jax.ShapeDtypeStruct((N, 1), f32),
            jax.ShapeDtypeStruct((N, 128), f32),
        ],
    )(x, w0, d0, d1)


def _t1_body(p0, p1, dinv, b0, w1, h1_ref, xs_ref):
    dv = dinv[...]
    h1 = jnp.maximum((p0[...] + p1[...]) * dv + b0[...], 0.0)
    h1_ref[...] = h1
    xs_ref[...] = jnp.dot(h1, w1[...], preferred_element_type=f32) * dv


def _t1(p0, p1, dinv, b0, w1):
    return pl.pallas_call(
        _t1_body,
        grid=(N // BLK,),
        in_specs=[_row_spec(128), _row_spec(128), _row_spec(1),
                  _rep_spec(1, 128), _rep_spec(128, 128)],
        out_specs=[_row_spec(128), _row_spec(128)],
        out_shape=[
            jax.ShapeDtypeStruct((N, 128), f32),
            jax.ShapeDtypeStruct((N, 128), f32),
        ],
    )(p0, p1, dinv, b0, w1)


def _t2_body(p0, p1, dinv, b1, h1, w2, xs_ref):
    dv = dinv[...]
    h2 = h1[...] + jnp.maximum((p0[...] + p1[...]) * dv + b1[...], 0.0)
    xs_ref[...] = jnp.dot(h2, w2[...], preferred_element_type=f32) * dv


def _t2(p0, p1, dinv, b1, h1, w2):
    return pl.pallas_call(
        _t2_body,
        grid=(N // BLK,),
        in_specs=[_row_spec(128), _row_spec(128), _row_spec(1),
                  _rep_spec(1, 128), _row_spec(128), _rep_spec(128, 64)],
        out_specs=[_row_spec(64)],
        out_shape=[jax.ShapeDtypeStruct((N, 64), f32)],
    )(p0, p1, dinv, b1, h1, w2)[0]


def _t3_body(p0, p1, dinv, b2, wg, acomb, xwg_ref, aout_ref):
    dv = dinv[...]
    h3 = jnp.maximum((p0[...] + p1[...]) * dv + b2[...], 0.0)
    xwg = jnp.dot(h3, wg[...], preferred_element_type=f32)
    xwg_ref[...] = xwg
    aout_ref[...] = jnp.dot(xwg, acomb[...], preferred_element_type=f32)


def _t3(p0, p1, dinv, b2, wg, acomb):
    return pl.pallas_call(
        _t3_body,
        grid=(N // BLK,),
        in_specs=[_row_spec(64), _row_spec(64), _row_spec(1),
                  _rep_spec(1, 64), _rep_spec(64, 256), _rep_spec(256, 128)],
        out_specs=[_row_spec(256), _row_spec(128)],
        out_shape=[
            jax.ShapeDtypeStruct((N, 256), f32),
            jax.ShapeDtypeStruct((N, 128), f32),
        ],
    )(p0, p1, dinv, b2, wg, acomb)


def _t3b_body(asrc_ref, adst_ref, c_ref):
    amax = jnp.max(asrc_ref[...], axis=0, keepdims=True)
    x = amax + adst_ref[...]
    c_ref[...] = jnp.maximum(x, 0.0) + 0.2 * jnp.minimum(x, 0.0)


def _t3b(asrc, adst):
    return pl.pallas_call(
        _t3b_body,
        grid=(1,),
        in_specs=[_rep_spec(N, HEADS), _rep_spec(N, HEADS)],
        out_specs=[_rep_spec(N, HEADS)],
        out_shape=[jax.ShapeDtypeStruct((N, HEADS), f32)],
    )(asrc, adst)[0]


def _t4_body(d0_ref, d1_ref, r_ref):
    r_ref[...] = 1.0 / (d0_ref[...] + d1_ref[...] + 1e-16)


def _t4(d0, d1):
    r = HEADS * N_PAD // 128
    return pl.pallas_call(
        _t4_body,
        grid=(1,),
        in_specs=[_rep_spec(r, 128), _rep_spec(r, 128)],
        out_specs=[_rep_spec(r, 128)],
        out_shape=[jax.ShapeDtypeStruct((r, 128), f32)],
    )(d0, d1)[0]


def _t5_body(p0, p1, bg, wcomb, bcomb, h4_ref, ge_ref, bo_ref):
    h4 = p0[...] + p1[...] + bg[...]
    h4_ref[...] = h4

    @pl.when(pl.program_id(0) == 0)
    def _():
        ge_ref[...] = jnp.zeros_like(ge_ref)

    ge_ref[...] += jnp.sum(h4, axis=0, keepdims=True) * (1.0 / N)
    bo_ref[...] = jnp.dot(h4, wcomb[...], preferred_element_type=f32) + bcomb[...]


def _t5(p0, p1, bg, wcomb, bcomb):
    return pl.pallas_call(
        _t5_body,
        grid=(N // BLK,),
        in_specs=[_row_spec(64), _row_spec(64), _rep_spec(1, 64),
                  _rep_spec(64, 128), _rep_spec(1, 128)],
        out_specs=[_row_spec(64), _rep_spec(1, 64), _row_spec(128)],
        out_shape=[
            jax.ShapeDtypeStruct((N, 64), f32),
            jax.ShapeDtypeStruct((1, 64), f32),
            jax.ShapeDtypeStruct((N, 128), f32),
        ],
    )(p0, p1, bg, wcomb, bcomb)


def _pad_rows(t):
    return jnp.pad(t, ((0, N_PAD - N), (0, 0)))


def _flat_t(a):
    return jnp.pad(a.T, ((0, 0), (0, N_PAD - N))).reshape(-1)


def kernel(x, edge_index, W0, b0, W1, b1, W2, b2, Wg, att_src, att_dst, bg,
           Wb, bb, Wo, bo):
    e_in = edge_index.shape[1]
    etot = e_in + N
    cpw = -(-etot // (NW * CHUNK))
    cpw += cpw % 2
    epad = cpw * NW * CHUNK

    loop = jnp.arange(N, dtype=i32)
    fill = TRASH + (jnp.arange(epad - etot, dtype=i32) % (N_PAD - N))
    srcp = jnp.concatenate([edge_index[0], loop, fill])
    dstp = jnp.concatenate([edge_index[1], loop, fill])
    nch = epad // CHUNK
    pack = jnp.stack([srcp.reshape(nch, CHUNK), dstp.reshape(nch, CHUNK)],
                     axis=1).reshape(2 * nch, CHUNK)
    pack = jnp.concatenate([pack, jnp.zeros((2, CHUNK), i32)], axis=0)

    z1 = jnp.zeros((RPT,), f32)
    z128 = jnp.zeros((RPT, 128), f32)
    z64 = jnp.zeros((RPT, 64), f32)
    z4 = jnp.zeros((HEADS * RPT,), f32)

    deg = _make_deg(cpw)(dstp, z1)
    d0 = deg[0, :N].reshape(N, 1)
    d1 = deg[1, :N].reshape(N, 1)
    dinv, xw0s = _t0(x, W0, d0, d1)

    gcn128 = _make_gcn2(cpw)
    p = gcn128(pack, _pad_rows(xw0s), z128)
    h1, xw1s = _t1(p[0, :N], p[1, :N], dinv, b0.reshape(1, -1), W1)

    p = gcn128(pack, _pad_rows(xw1s), z128)
    xw2s = _t2(p[0, :N], p[1, :N], dinv, b1.reshape(1, -1), h1, W2)

    xw2s_wide = jnp.pad(xw2s, ((0, 0), (0, 64)))
    p = gcn128(pack, _pad_rows(xw2s_wide), z128)
    p = p[:, :, :64]

    acomb = jnp.zeros((256, 128), f32)
    for h in range(HEADS):
        acomb = acomb.at[h * 64:(h + 1) * 64, h].set(att_src[h])
        acomb = acomb.at[h * 64:(h + 1) * 64, HEADS + h].set(att_dst[h])
    xwg, aout = _t3(p[0, :N], p[1, :N], dinv, b2.reshape(1, -1), Wg, acomb)
    asrc = aout[:, :HEADS]
    adst = aout[:, HEADS:2 * HEADS]
    cvals = _t3b(asrc, adst)

    den, ex = _make_gat_a(epad, cpw)(
        srcp, dstp, _flat_t(asrc), _flat_t(adst), _flat_t(cvals), z4
    )
    r = HEADS * N_PAD // 128
    rden = _t4(den[0].reshape(r, 128), den[1].reshape(r, 128)).reshape(-1)

    pg = _make_gat_b(epad, cpw)(srcp, dstp, ex, rden, _pad_rows(xwg), z128)

    wcomb = jnp.zeros((64, 128), f32)
    wcomb = wcomb.at[:, :2].set(Wb)
    wcomb = wcomb.at[:, 2:8].set(Wo)
    bcomb = jnp.zeros((1, 128), f32)
    bcomb = bcomb.at[0, :2].set(bb)
    bcomb = bcomb.at[0, 2:8].set(bo)

    h4, ge, bo_full = _t5(pg[0, :N, :64], pg[1, :N, :64], bg.reshape(1, -1), wcomb, bcomb)
    return h4, ge, bo_full[:, :2], bo_full[:, 2:8]


# speedup vs baseline: 2.1211x; 1.5559x over previous
import functools

import jax
import jax.numpy as jnp
from jax import lax
from jax.experimental import pallas as pl
from jax.experimental.pallas import tpu as pltpu
from jax.experimental.pallas import tpu_sc as plsc

f32 = jnp.float32
i32 = jnp.int32

N = 10000
HEADS = 4
NC = 2
NS = 16
NW = NC * NS
CHUNK = 128
N_PAD = 10240
TRASH = N
RPT = N_PAD // NS
BLK = 1000


def _mesh():
    return plsc.VectorSubcoreMesh(
        core_axis_name="c", subcore_axis_name="s", num_cores=NC, num_subcores=NS
    )


@functools.lru_cache(maxsize=None)
def _make_deg(cpw):
    @functools.partial(
        pl.kernel,
        out_type=jax.ShapeDtypeStruct((NC, N_PAD), f32),
        mesh=_mesh(),
        scratch_types=[
            pltpu.VMEM_SHARED((N_PAD,), f32),
            pltpu.VMEM((CHUNK,), i32),
            pltpu.VMEM((CHUNK,), f32),
        ],
    )
    def deg_kernel(dst_hbm, zeros_hbm, out_hbm, acc_sh, dstidx, ones_v):
        c = lax.axis_index("c")
        s = lax.axis_index("s")
        w = c * NS + s
        for j in range(CHUNK // 16):
            ones_v[pl.ds(j * 16, 16)] = jnp.ones((16,), f32)
        pltpu.sync_copy(zeros_hbm, acc_sh.at[pl.ds(s * RPT, RPT)])
        plsc.subcore_barrier()

        def body(k, carry):
            base = (w * cpw + k) * CHUNK
            pltpu.sync_copy(dst_hbm.at[pl.ds(base, CHUNK)], dstidx)
            pltpu.sync_copy(ones_v, acc_sh.at[dstidx], add=True)
            return carry

        lax.fori_loop(0, cpw, body, 0)
        plsc.subcore_barrier()
        pltpu.sync_copy(
            acc_sh.at[pl.ds(s * RPT, RPT)], out_hbm.at[c, pl.ds(s * RPT, RPT)]
        )

    return deg_kernel


@functools.lru_cache(maxsize=None)
def _make_gcn2(cpw):
    @functools.partial(
        pl.kernel,
        out_type=jax.ShapeDtypeStruct((NC, N_PAD, 128), f32),
        mesh=_mesh(),
        scratch_types=[
            pltpu.VMEM_SHARED((N_PAD, 128), f32),
            pltpu.VMEM((2, CHUNK), i32),
            pltpu.VMEM((2, CHUNK), i32),
            pltpu.VMEM((CHUNK, 128), f32),
            pltpu.VMEM((CHUNK, 128), f32),
            pltpu.SemaphoreType.DMA,
            pltpu.SemaphoreType.DMA,
        ],
    )
    def gcn_kernel(pack_hbm, tab_hbm, zeros_hbm, out_hbm,
                   acc_sh, idxa, idxb, rowsa, rowsb, sema, semb):
        c = lax.axis_index("c")
        s = lax.axis_index("s")
        w = c * NS + s
        c0 = w * cpw
        pltpu.sync_copy(zeros_hbm, acc_sh.at[pl.ds(s * RPT, RPT)])
        plsc.subcore_barrier()
        pltpu.sync_copy(pack_hbm.at[pl.ds(2 * c0, 2)], idxa)
        pltpu.async_copy(tab_hbm.at[idxa.at[0]], rowsa, sema)

        def body(k2, carry):
            ck = c0 + 2 * k2
            pltpu.make_async_copy(tab_hbm.at[idxa.at[0]], rowsa, sema).wait()
            pltpu.sync_copy(pack_hbm.at[pl.ds(2 * (ck + 1), 2)], idxb)
            pltpu.async_copy(tab_hbm.at[idxb.at[0]], rowsb, semb)
            pltpu.sync_copy(rowsa, acc_sh.at[idxa.at[1]], add=True)
            pltpu.make_async_copy(tab_hbm.at[idxb.at[0]], rowsb, semb).wait()
            pltpu.sync_copy(pack_hbm.at[pl.ds(2 * (ck + 2), 2)], idxa)
            pltpu.async_copy(tab_hbm.at[idxa.at[0]], rowsa, sema)
            pltpu.sync_copy(rowsb, acc_sh.at[idxb.at[1]], add=True)
            return carry

        lax.fori_loop(0, cpw // 2, body, 0)
        pltpu.make_async_copy(tab_hbm.at[idxa.at[0]], rowsa, sema).wait()
        plsc.subcore_barrier()
        pltpu.sync_copy(
            acc_sh.at[pl.ds(s * RPT, RPT)], out_hbm.at[c, pl.ds(s * RPT, RPT)]
        )

    return gcn_kernel


@functools.lru_cache(maxsize=None)
def _make_gcn(d, cpw):
    @functools.partial(
        pl.kernel,
        out_type=jax.ShapeDtypeStruct((NC, N_PAD, d), f32),
        mesh=_mesh(),
        scratch_types=[
            pltpu.VMEM_SHARED((N_PAD, d), f32),
            pltpu.VMEM((CHUNK,), i32),
            pltpu.VMEM((CHUNK,), i32),
            pltpu.VMEM((CHUNK, d), f32),
            pltpu.SemaphoreType.DMA,
        ],
    )
    def gcn_kernel(src_hbm, dst_hbm, tab_hbm, zeros_hbm, out_hbm,
                   acc_sh, srcidx, dstidx, rows, sem):
        c = lax.axis_index("c")
        s = lax.axis_index("s")
        w = c * NS + s
        pltpu.sync_copy(zeros_hbm, acc_sh.at[pl.ds(s * RPT, RPT)])
        plsc.subcore_barrier()

        def body(k, carry):
            base = (w * cpw + k) * CHUNK
            pltpu.sync_copy(src_hbm.at[pl.ds(base, CHUNK)], srcidx)
            pltpu.sync_copy(dst_hbm.at[pl.ds(base, CHUNK)], dstidx)
            pltpu.async_copy(tab_hbm.at[srcidx], rows, sem).wait()
            pltpu.sync_copy(rows, acc_sh.at[dstidx], add=True)
            return carry

        lax.fori_loop(0, cpw, body, 0)
        plsc.subcore_barrier()
        pltpu.sync_copy(
            acc_sh.at[pl.ds(s * RPT, RPT)], out_hbm.at[c, pl.ds(s * RPT, RPT)]
        )

    return gcn_kernel


@functools.lru_cache(maxsize=None)
def _make_gat_a(epad, cpw):
    @functools.partial(
        pl.kernel,
        out_type=(
            jax.ShapeDtypeStruct((NC, HEADS * N_PAD), f32),
            jax.ShapeDtypeStruct((HEADS * (epad // CHUNK), CHUNK), f32),
        ),
        mesh=_mesh(),
        scratch_types=[
            pltpu.VMEM_SHARED((HEADS * N_PAD,), f32),
            pltpu.VMEM((CHUNK,), i32),
            pltpu.VMEM((CHUNK,), i32),
            pltpu.VMEM((HEADS, CHUNK), i32),
            pltpu.VMEM((HEADS, CHUNK), i32),
            pltpu.VMEM((HEADS, CHUNK), f32),
            pltpu.VMEM((HEADS, CHUNK), f32),
            pltpu.VMEM((HEADS, CHUNK), f32),
            pltpu.VMEM((HEADS, CHUNK), f32),
            pltpu.SemaphoreType.DMA,
        ],
    )
    def gat_a_kernel(src_hbm, dst_hbm, at_hbm, dt_hbm, ct_hbm, zeros_hbm,
                     den_hbm, ex_hbm,
                     acc_sh, srcidx, dstidx, sidx, didx, av, dv, cv, exv, sem):
        c = lax.axis_index("c")
        s = lax.axis_index("s")
        w = c * NS + s
        pltpu.sync_copy(zeros_hbm, acc_sh.at[pl.ds(s * (HEADS * RPT), HEADS * RPT)])
        plsc.subcore_barrier()

        def body(k, carry):
            ck = w * cpw + k
            base = ck * CHUNK
            pltpu.sync_copy(src_hbm.at[pl.ds(base, CHUNK)], srcidx)
            pltpu.sync_copy(dst_hbm.at[pl.ds(base, CHUNK)], dstidx)
            for h in range(HEADS):
                for j in range(CHUNK // 16):
                    sl = pl.ds(j * 16, 16)
                    sidx[h, sl] = srcidx[sl] + (h * N_PAD)
                    didx[h, sl] = dstidx[sl] + (h * N_PAD)
            for h in range(HEADS):
                pltpu.async_copy(at_hbm.at[sidx.at[h]], av.at[h], sem)
                pltpu.async_copy(dt_hbm.at[didx.at[h]], dv.at[h], sem)
                pltpu.async_copy(ct_hbm.at[didx.at[h]], cv.at[h], sem)
            for h in range(HEADS):
                pltpu.make_async_copy(at_hbm.at[sidx.at[h]], av.at[h], sem).wait()
                pltpu.make_async_copy(dt_hbm.at[didx.at[h]], dv.at[h], sem).wait()
                pltpu.make_async_copy(ct_hbm.at[didx.at[h]], cv.at[h], sem).wait()
            for h in range(HEADS):
                for j in range(CHUNK // 16):
                    sl = pl.ds(j * 16, 16)
                    x = av[h, sl] + dv[h, sl]
                    e = jnp.maximum(x, 0.0) + 0.2 * jnp.minimum(x, 0.0)
                    exv[h, sl] = jnp.exp(e - cv[h, sl])
            pltpu.sync_copy(exv, ex_hbm.at[pl.ds(HEADS * ck, HEADS)])
            for h in range(HEADS):
                pltpu.async_copy(exv.at[h], acc_sh.at[didx.at[h]], sem, add=True)
            for h in range(HEADS):
                pltpu.make_async_copy(exv.at[h], acc_sh.at[didx.at[h]], sem).wait()
            return carry

        lax.fori_loop(0, cpw, body, 0)
        plsc.subcore_barrier()
        pltpu.sync_copy(
            acc_sh.at[pl.ds(s * (HEADS * RPT), HEADS * RPT)],
            den_hbm.at[c, pl.ds(s * (HEADS * RPT), HEADS * RPT)],
        )

    return gat_a_kernel


@functools.lru_cache(maxsize=None)
def _make_gat_b(epad, cpw):
    QE = CHUNK // 4

    @functools.partial(
        pl.kernel,
        out_type=jax.ShapeDtypeStruct((NC, N_PAD, 128), f32),
        mesh=_mesh(),
        scratch_types=[
            pltpu.VMEM_SHARED((N_PAD, 128), f32),
            pltpu.VMEM((CHUNK,), i32),
            pltpu.VMEM((HEADS, CHUNK), i32),
            pltpu.VMEM((HEADS, CHUNK), f32),
            pltpu.VMEM((HEADS, CHUNK), f32),
            pltpu.VMEM((HEADS * CHUNK,), f32),
            pltpu.VMEM((QE,), i32),
            pltpu.VMEM((QE,), i32),
            pltpu.VMEM((QE, HEADS * 64), f32),
            pltpu.VMEM((QE, HEADS * 64), f32),
            pltpu.VMEM((CHUNK, 128), f32),
            pltpu.SemaphoreType.DMA,
            pltpu.SemaphoreType.DMA,
            pltpu.SemaphoreType.DMA,
        ],
    )
    def gat_b_kernel(src_hbm, dst_hbm, ex_hbm, rdt_hbm, xwg_hbm, zeros_hbm,
                     out_hbm, acc_sh, dstidx, didx, exv, rdv, alv,
                     sidx0, sidx1, rows0, rows1, outc, sem, semr0, semr1):
        c = lax.axis_index("c")
        s = lax.axis_index("s")
        w = c * NS + s
        pltpu.sync_copy(zeros_hbm, acc_sh.at[pl.ds(s * RPT, RPT)])

        def zrow(e, carry):
            for j in range(4):
                outc[e, pl.ds(64 + j * 16, 16)] = jnp.zeros((16,), f32)
            return carry

        lax.fori_loop(0, CHUNK, zrow, 0)
        plsc.subcore_barrier()

        sidx = (sidx0, sidx1)
        rows = (rows0, rows1)
        semr = (semr0, semr1)

        def body(k, carry):
            ck = w * cpw + k
            base = ck * CHUNK
            pltpu.sync_copy(src_hbm.at[pl.ds(base, QE)], sidx0)
            pltpu.async_copy(xwg_hbm.at[sidx0], rows0, semr0)
            pltpu.sync_copy(dst_hbm.at[pl.ds(base, CHUNK)], dstidx)
            for h in range(HEADS):
                for j in range(CHUNK // 16):
                    sl = pl.ds(j * 16, 16)
                    didx[h, sl] = dstidx[sl] + (h * N_PAD)
            pltpu.sync_copy(ex_hbm.at[pl.ds(HEADS * ck, HEADS)], exv)
            for h in range(HEADS):
                pltpu.async_copy(rdt_hbm.at[didx.at[h]], rdv.at[h], sem)
            for h in range(HEADS):
                pltpu.make_async_copy(rdt_hbm.at[didx.at[h]], rdv.at[h], sem).wait()
            for h in range(HEADS):
                for j in range(CHUNK // 16):
                    sl = pl.ds(j * 16, 16)
                    alv[pl.ds(h * CHUNK + j * 16, 16)] = (
                        exv[h, sl] * rdv[h, sl] * 0.25
                    )
            for q in range(4):
                b = q % 2
                nb = 1 - b
                pltpu.make_async_copy(
                    xwg_hbm.at[sidx[b]], rows[b], semr[b]).wait()
                if q < 3:
                    pltpu.sync_copy(
                        src_hbm.at[pl.ds(base + (q + 1) * QE, QE)], sidx[nb])
                    pltpu.async_copy(xwg_hbm.at[sidx[nb]], rows[nb], semr[nb])

                def edge(e, carry2, q=q, b=b):
                    eo = e + q * QE
                    e16 = (eo // 16) * 16
                    lanev = jnp.full((16,), eo - e16, dtype=i32)
                    acc = [jnp.zeros((16,), f32) for _ in range(4)]
                    for h in range(HEADS):
                        vv = alv[pl.ds(h * CHUNK + e16, 16)]
                        ah = lax.gather(
                            vv, lanev[:, None],
                            lax.GatherDimensionNumbers(
                                offset_dims=(), collapsed_slice_dims=(0,),
                                start_index_map=(0,)),
                            (1,),
                            mode=lax.GatherScatterMode.PROMISE_IN_BOUNDS,
                        )
                        for j in range(4):
                            acc[j] = acc[j] + ah * rows[b][e, pl.ds(h * 64 + j * 16, 16)]
                    for j in range(4):
                        outc[eo, pl.ds(j * 16, 16)] = acc[j]
                    return carry2

                lax.fori_loop(0, QE, edge, 0)
            pltpu.sync_copy(outc, acc_sh.at[dstidx], add=True)
            return carry

        lax.fori_loop(0, cpw, body, 0)
        plsc.subcore_barrier()
        pltpu.sync_copy(
            acc_sh.at[pl.ds(s * RPT, RPT)], out_hbm.at[c, pl.ds(s * RPT, RPT)]
        )

    return gat_b_kernel


def _row_spec(dd):
    return pl.BlockSpec((BLK, dd), lambda i: (i, 0))


def _rep_spec(r, dd):
    return pl.BlockSpec((r, dd), lambda i: (0, 0))


def _t0_body(x_ref, w_ref, d0_ref, d1_ref, dinv_ref, xs_ref):
    dinv = lax.rsqrt(d0_ref[...] + d1_ref[...])
    dinv_ref[...] = dinv
    xs_ref[...] = jnp.dot(x_ref[...], w_ref[...], preferred_element_type=f32) * dinv


def _t0(x, w0, d0, d1):
    return pl.pallas_call(
        _t0_body,
        grid=(N // BLK,),
        in_specs=[_row_spec(128), _rep_spec(128, 128), _row_spec(1), _row_spec(1)],
        out_specs=[_row_spec(1), _row_spec(128)],
        out_shape=[
            jax.ShapeDtypeStruct((N, 1), f32),
            jax.ShapeDtypeStruct((N, 128), f32),
        ],
    )(x, w0, d0, d1)


def _t1_body(p0, p1, dinv, b0, w1, h1_ref, xs_ref):
    dv = dinv[...]
    h1 = jnp.maximum((p0[...] + p1[...]) * dv + b0[...], 0.0)
    h1_ref[...] = h1
    xs_ref[...] = jnp.dot(h1, w1[...], preferred_element_type=f32) * dv


def _t1(p0, p1, dinv, b0, w1):
    return pl.pallas_call(
        _t1_body,
        grid=(N // BLK,),
        in_specs=[_row_spec(128), _row_spec(128), _row_spec(1),
                  _rep_spec(1, 128), _rep_spec(128, 128)],
        out_specs=[_row_spec(128), _row_spec(128)],
        out_shape=[
            jax.ShapeDtypeStruct((N, 128), f32),
            jax.ShapeDtypeStruct((N, 128), f32),
        ],
    )(p0, p1, dinv, b0, w1)


def _t2_body(p0, p1, dinv, b1, h1, w2, xs_ref):
    dv = dinv[...]
    h2 = h1[...] + jnp.maximum((p0[...] + p1[...]) * dv + b1[...], 0.0)
    xs_ref[...] = jnp.dot(h2, w2[...], preferred_element_type=f32) * dv


def _t2(p0, p1, dinv, b1, h1, w2):
    return pl.pallas_call(
        _t2_body,
        grid=(N // BLK,),
        in_specs=[_row_spec(128), _row_spec(128), _row_spec(1),
                  _rep_spec(1, 128), _row_spec(128), _rep_spec(128, 64)],
        out_specs=[_row_spec(64)],
        out_shape=[jax.ShapeDtypeStruct((N, 64), f32)],
    )(p0, p1, dinv, b1, h1, w2)[0]


def _t3_body(p0, p1, dinv, b2, wg, acomb, xwg_ref, aout_ref):
    dv = dinv[...]
    h3 = jnp.maximum((p0[...] + p1[...]) * dv + b2[...], 0.0)
    xwg = jnp.dot(h3, wg[...], preferred_element_type=f32)
    xwg_ref[...] = xwg
    aout_ref[...] = jnp.dot(xwg, acomb[...], preferred_element_type=f32)


def _t3(p0, p1, dinv, b2, wg, acomb):
    return pl.pallas_call(
        _t3_body,
        grid=(N // BLK,),
        in_specs=[_row_spec(64), _row_spec(64), _row_spec(1),
                  _rep_spec(1, 64), _rep_spec(64, 256), _rep_spec(256, 128)],
        out_specs=[_row_spec(256), _row_spec(128)],
        out_shape=[
            jax.ShapeDtypeStruct((N, 256), f32),
            jax.ShapeDtypeStruct((N, 128), f32),
        ],
    )(p0, p1, dinv, b2, wg, acomb)


def _t3b_body(asrc_ref, adst_ref, c_ref):
    amax = jnp.max(asrc_ref[...], axis=0, keepdims=True)
    x = amax + adst_ref[...]
    c_ref[...] = jnp.maximum(x, 0.0) + 0.2 * jnp.minimum(x, 0.0)


def _t3b(asrc, adst):
    return pl.pallas_call(
        _t3b_body,
        grid=(1,),
        in_specs=[_rep_spec(N, HEADS), _rep_spec(N, HEADS)],
        out_specs=[_rep_spec(N, HEADS)],
        out_shape=[jax.ShapeDtypeStruct((N, HEADS), f32)],
    )(asrc, adst)[0]


def _t4_body(d0_ref, d1_ref, r_ref):
    r_ref[...] = 1.0 / (d0_ref[...] + d1_ref[...] + 1e-16)


def _t4(d0, d1):
    r = HEADS * N_PAD // 128
    return pl.pallas_call(
        _t4_body,
        grid=(1,),
        in_specs=[_rep_spec(r, 128), _rep_spec(r, 128)],
        out_specs=[_rep_spec(r, 128)],
        out_shape=[jax.ShapeDtypeStruct((r, 128), f32)],
    )(d0, d1)[0]


def _t5_body(p0, p1, bg, wcomb, bcomb, h4_ref, ge_ref, bo_ref):
    h4 = p0[...] + p1[...] + bg[...]
    h4_ref[...] = h4

    @pl.when(pl.program_id(0) == 0)
    def _():
        ge_ref[...] = jnp.zeros_like(ge_ref)

    ge_ref[...] += jnp.sum(h4, axis=0, keepdims=True) * (1.0 / N)
    bo_ref[...] = jnp.dot(h4, wcomb[...], preferred_element_type=f32) + bcomb[...]


def _t5(p0, p1, bg, wcomb, bcomb):
    return pl.pallas_call(
        _t5_body,
        grid=(N // BLK,),
        in_specs=[_row_spec(64), _row_spec(64), _rep_spec(1, 64),
                  _rep_spec(64, 128), _rep_spec(1, 128)],
        out_specs=[_row_spec(64), _rep_spec(1, 64), _row_spec(128)],
        out_shape=[
            jax.ShapeDtypeStruct((N, 64), f32),
            jax.ShapeDtypeStruct((1, 64), f32),
            jax.ShapeDtypeStruct((N, 128), f32),
        ],
    )(p0, p1, bg, wcomb, bcomb)


def _pad_rows(t):
    return jnp.pad(t, ((0, N_PAD - N), (0, 0)))


def _flat_t(a):
    return jnp.pad(a.T, ((0, 0), (0, N_PAD - N))).reshape(-1)


def kernel(x, edge_index, W0, b0, W1, b1, W2, b2, Wg, att_src, att_dst, bg,
           Wb, bb, Wo, bo):
    e_in = edge_index.shape[1]
    etot = e_in + N
    cpw = -(-etot // (NW * CHUNK))
    cpw += cpw % 2
    epad = cpw * NW * CHUNK

    loop = jnp.arange(N, dtype=i32)
    fill = TRASH + (jnp.arange(epad - etot, dtype=i32) % (N_PAD - N))
    srcp = jnp.concatenate([edge_index[0], loop, fill])
    dstp = jnp.concatenate([edge_index[1], loop, fill])
    nch = epad // CHUNK
    pack = jnp.stack([srcp.reshape(nch, CHUNK), dstp.reshape(nch, CHUNK)],
                     axis=1).reshape(2 * nch, CHUNK)
    pack = jnp.concatenate([pack, jnp.zeros((2, CHUNK), i32)], axis=0)

    z1 = jnp.zeros((RPT,), f32)
    z128 = jnp.zeros((RPT, 128), f32)
    z64 = jnp.zeros((RPT, 64), f32)
    z4 = jnp.zeros((HEADS * RPT,), f32)

    deg = _make_deg(cpw)(dstp, z1)
    d0 = deg[0, :N].reshape(N, 1)
    d1 = deg[1, :N].reshape(N, 1)
    dinv, xw0s = _t0(x, W0, d0, d1)

    gcn128 = _make_gcn2(cpw)
    p = gcn128(pack, _pad_rows(xw0s), z128)
    h1, xw1s = _t1(p[0, :N], p[1, :N], dinv, b0.reshape(1, -1), W1)

    p = gcn128(pack, _pad_rows(xw1s), z128)
    xw2s = _t2(p[0, :N], p[1, :N], dinv, b1.reshape(1, -1), h1, W2)

    xw2s_wide = jnp.pad(xw2s, ((0, 0), (0, 64)))
    p = gcn128(pack, _pad_rows(xw2s_wide), z128)
    p = p[:, :, :64]

    acomb = jnp.zeros((256, 128), f32)
    for h in range(HEADS):
        acomb = acomb.at[h * 64:(h + 1) * 64, h].set(att_src[h])
        acomb = acomb.at[h * 64:(h + 1) * 64, HEADS + h].set(att_dst[h])
    xwg, aout = _t3(p[0, :N], p[1, :N], dinv, b2.reshape(1, -1), Wg, acomb)
    asrc = aout[:, :HEADS]
    adst = aout[:, HEADS:2 * HEADS]
    cvals = _t3b(asrc, adst)

    den, ex = _make_gat_a(epad, cpw)(
        srcp, dstp, _flat_t(asrc), _flat_t(adst), _flat_t(cvals), z4
    )
    r = HEADS * N_PAD // 128
    rden = _t4(den[0].reshape(r, 128), den[1].reshape(r, 128)).reshape(-1)

    pg = _make_gat_b(epad, cpw)(srcp, dstp, ex, rden, _pad_rows(xwg), z128)

    wcomb = jnp.zeros((64, 128), f32)
    wcomb = wcomb.at[:, :2].set(Wb)
    wcomb = wcomb.at[:, 2:8].set(Wo)
    bcomb = jnp.zeros((1, 128), f32)
    bcomb = bcomb.at[0, :2].set(bb)
    bcomb = bcomb.at[0, 2:8].set(bo)

    h4, ge, bo_full = _t5(pg[0, :N, :64], pg[1, :N, :64], bg.reshape(1, -1), wcomb, bcomb)
    return h4, ge, bo_full[:, :2], bo_full[:, 2:8]


# speedup vs baseline: 2.1243x; 1.0015x over previous
import functools

import jax
import jax.numpy as jnp
from jax import lax
from jax.experimental import pallas as pl
from jax.experimental.pallas import tpu as pltpu
from jax.experimental.pallas import tpu_sc as plsc

f32 = jnp.float32
i32 = jnp.int32

N = 10000
HEADS = 4
NC = 2
NS = 16
NW = NC * NS
CHUNK = 128
N_PAD = 10240
TRASH = N
RPT = N_PAD // NS
BLK = 1000


def _mesh():
    return plsc.VectorSubcoreMesh(
        core_axis_name="c", subcore_axis_name="s", num_cores=NC, num_subcores=NS
    )


@functools.lru_cache(maxsize=None)
def _make_deg(cpw):
    @functools.partial(
        pl.kernel,
        out_type=jax.ShapeDtypeStruct((NC, N_PAD), f32),
        mesh=_mesh(),
        scratch_types=[
            pltpu.VMEM_SHARED((N_PAD,), f32),
            pltpu.VMEM((CHUNK,), i32),
            pltpu.VMEM((CHUNK,), f32),
        ],
    )
    def deg_kernel(dst_hbm, zeros_hbm, out_hbm, acc_sh, dstidx, ones_v):
        c = lax.axis_index("c")
        s = lax.axis_index("s")
        w = c * NS + s
        for j in range(CHUNK // 16):
            ones_v[pl.ds(j * 16, 16)] = jnp.ones((16,), f32)
        pltpu.sync_copy(zeros_hbm, acc_sh.at[pl.ds(s * RPT, RPT)])
        plsc.subcore_barrier()

        def body(k, carry):
            base = (w * cpw + k) * CHUNK
            pltpu.sync_copy(dst_hbm.at[pl.ds(base, CHUNK)], dstidx)
            pltpu.sync_copy(ones_v, acc_sh.at[dstidx], add=True)
            return carry

        lax.fori_loop(0, cpw, body, 0)
        plsc.subcore_barrier()
        pltpu.sync_copy(
            acc_sh.at[pl.ds(s * RPT, RPT)], out_hbm.at[c, pl.ds(s * RPT, RPT)]
        )

    return deg_kernel


@functools.lru_cache(maxsize=None)
def _make_gcn2(cpw):
    @functools.partial(
        pl.kernel,
        out_type=jax.ShapeDtypeStruct((NC, N_PAD, 128), f32),
        mesh=_mesh(),
        scratch_types=[
            pltpu.VMEM_SHARED((N_PAD, 128), f32),
            pltpu.VMEM((2, CHUNK), i32),
            pltpu.VMEM((2, CHUNK), i32),
            pltpu.VMEM((CHUNK, 128), f32),
            pltpu.VMEM((CHUNK, 128), f32),
            pltpu.SemaphoreType.DMA,
            pltpu.SemaphoreType.DMA,
        ],
    )
    def gcn_kernel(pack_hbm, tab_hbm, zeros_hbm, out_hbm,
                   acc_sh, idxa, idxb, rowsa, rowsb, sema, semb):
        c = lax.axis_index("c")
        s = lax.axis_index("s")
        w = c * NS + s
        c0 = w * cpw
        pltpu.sync_copy(zeros_hbm, acc_sh.at[pl.ds(s * RPT, RPT)])
        plsc.subcore_barrier()
        pltpu.sync_copy(pack_hbm.at[pl.ds(2 * c0, 2)], idxa)
        pltpu.async_copy(tab_hbm.at[idxa.at[0]], rowsa, sema)

        def body(k2, carry):
            ck = c0 + 2 * k2
            pltpu.make_async_copy(tab_hbm.at[idxa.at[0]], rowsa, sema).wait()
            pltpu.sync_copy(pack_hbm.at[pl.ds(2 * (ck + 1), 2)], idxb)
            pltpu.async_copy(tab_hbm.at[idxb.at[0]], rowsb, semb)
            pltpu.sync_copy(rowsa, acc_sh.at[idxa.at[1]], add=True)
            pltpu.make_async_copy(tab_hbm.at[idxb.at[0]], rowsb, semb).wait()
            pltpu.sync_copy(pack_hbm.at[pl.ds(2 * (ck + 2), 2)], idxa)
            pltpu.async_copy(tab_hbm.at[idxa.at[0]], rowsa, sema)
            pltpu.sync_copy(rowsb, acc_sh.at[idxb.at[1]], add=True)
            return carry

        lax.fori_loop(0, cpw // 2, body, 0)
        pltpu.make_async_copy(tab_hbm.at[idxa.at[0]], rowsa, sema).wait()
        plsc.subcore_barrier()
        pltpu.sync_copy(
            acc_sh.at[pl.ds(s * RPT, RPT)], out_hbm.at[c, pl.ds(s * RPT, RPT)]
        )

    return gcn_kernel


@functools.lru_cache(maxsize=None)
def _make_gcn(d, cpw):
    @functools.partial(
        pl.kernel,
        out_type=jax.ShapeDtypeStruct((NC, N_PAD, d), f32),
        mesh=_mesh(),
        scratch_types=[
            pltpu.VMEM_SHARED((N_PAD, d), f32),
            pltpu.VMEM((CHUNK,), i32),
            pltpu.VMEM((CHUNK,), i32),
            pltpu.VMEM((CHUNK, d), f32),
            pltpu.SemaphoreType.DMA,
        ],
    )
    def gcn_kernel(src_hbm, dst_hbm, tab_hbm, zeros_hbm, out_hbm,
                   acc_sh, srcidx, dstidx, rows, sem):
        c = lax.axis_index("c")
        s = lax.axis_index("s")
        w = c * NS + s
        pltpu.sync_copy(zeros_hbm, acc_sh.at[pl.ds(s * RPT, RPT)])
        plsc.subcore_barrier()

        def body(k, carry):
            base = (w * cpw + k) * CHUNK
            pltpu.sync_copy(src_hbm.at[pl.ds(base, CHUNK)], srcidx)
            pltpu.sync_copy(dst_hbm.at[pl.ds(base, CHUNK)], dstidx)
            pltpu.async_copy(tab_hbm.at[srcidx], rows, sem).wait()
            pltpu.sync_copy(rows, acc_sh.at[dstidx], add=True)
            return carry

        lax.fori_loop(0, cpw, body, 0)
        plsc.subcore_barrier()
        pltpu.sync_copy(
            acc_sh.at[pl.ds(s * RPT, RPT)], out_hbm.at[c, pl.ds(s * RPT, RPT)]
        )

    return gcn_kernel


@functools.lru_cache(maxsize=None)
def _make_gat_a(epad, cpw):
    @functools.partial(
        pl.kernel,
        out_type=(
            jax.ShapeDtypeStruct((NC, HEADS * N_PAD), f32),
            jax.ShapeDtypeStruct((HEADS * (epad // CHUNK), CHUNK), f32),
        ),
        mesh=_mesh(),
        scratch_types=[
            pltpu.VMEM_SHARED((HEADS * N_PAD,), f32),
            pltpu.VMEM((CHUNK,), i32),
            pltpu.VMEM((CHUNK,), i32),
            pltpu.VMEM((HEADS, CHUNK), i32),
            pltpu.VMEM((HEADS, CHUNK), i32),
            pltpu.VMEM((HEADS, CHUNK), f32),
            pltpu.VMEM((HEADS, CHUNK), f32),
            pltpu.VMEM((HEADS, CHUNK), f32),
            pltpu.VMEM((HEADS, CHUNK), f32),
            pltpu.SemaphoreType.DMA,
        ],
    )
    def gat_a_kernel(src_hbm, dst_hbm, at_hbm, dt_hbm, ct_hbm, zeros_hbm,
                     den_hbm, ex_hbm,
                     acc_sh, srcidx, dstidx, sidx, didx, av, dv, cv, exv, sem):
        c = lax.axis_index("c")
        s = lax.axis_index("s")
        w = c * NS + s
        pltpu.sync_copy(zeros_hbm, acc_sh.at[pl.ds(s * (HEADS * RPT), HEADS * RPT)])
        plsc.subcore_barrier()

        def body(k, carry):
            ck = w * cpw + k
            base = ck * CHUNK
            pltpu.sync_copy(src_hbm.at[pl.ds(base, CHUNK)], srcidx)
            pltpu.sync_copy(dst_hbm.at[pl.ds(base, CHUNK)], dstidx)
            for h in range(HEADS):
                for j in range(CHUNK // 16):
                    sl = pl.ds(j * 16, 16)
                    sidx[h, sl] = srcidx[sl] + (h * N_PAD)
                    didx[h, sl] = dstidx[sl] + (h * N_PAD)
            for h in range(HEADS):
                pltpu.async_copy(at_hbm.at[sidx.at[h]], av.at[h], sem)
                pltpu.async_copy(dt_hbm.at[didx.at[h]], dv.at[h], sem)
                pltpu.async_copy(ct_hbm.at[didx.at[h]], cv.at[h], sem)
            for h in range(HEADS):
                pltpu.make_async_copy(at_hbm.at[sidx.at[h]], av.at[h], sem).wait()
                pltpu.make_async_copy(dt_hbm.at[didx.at[h]], dv.at[h], sem).wait()
                pltpu.make_async_copy(ct_hbm.at[didx.at[h]], cv.at[h], sem).wait()
            for h in range(HEADS):
                for j in range(CHUNK // 16):
                    sl = pl.ds(j * 16, 16)
                    x = av[h, sl] + dv[h, sl]
                    e = jnp.maximum(x, 0.0) + 0.2 * jnp.minimum(x, 0.0)
                    exv[h, sl] = jnp.exp(e - cv[h, sl])
            pltpu.sync_copy(exv, ex_hbm.at[pl.ds(HEADS * ck, HEADS)])
            for h in range(HEADS):
                pltpu.async_copy(exv.at[h], acc_sh.at[didx.at[h]], sem, add=True)
            for h in range(HEADS):
                pltpu.make_async_copy(exv.at[h], acc_sh.at[didx.at[h]], sem).wait()
            return carry

        lax.fori_loop(0, cpw, body, 0)
        plsc.subcore_barrier()
        pltpu.sync_copy(
            acc_sh.at[pl.ds(s * (HEADS * RPT), HEADS * RPT)],
            den_hbm.at[c, pl.ds(s * (HEADS * RPT), HEADS * RPT)],
        )

    return gat_a_kernel


@functools.lru_cache(maxsize=None)
def _make_gat_b(epad, cpw):
    QE = CHUNK // 4

    @functools.partial(
        pl.kernel,
        out_type=jax.ShapeDtypeStruct((NC, N_PAD, 128), f32),
        mesh=_mesh(),
        scratch_types=[
            pltpu.VMEM_SHARED((N_PAD, 128), f32),
            pltpu.VMEM((CHUNK,), i32),
            pltpu.VMEM((HEADS, CHUNK), i32),
            pltpu.VMEM((HEADS, CHUNK), f32),
            pltpu.VMEM((HEADS, CHUNK), f32),
            pltpu.VMEM((HEADS * CHUNK,), f32),
            pltpu.VMEM((QE,), i32),
            pltpu.VMEM((QE,), i32),
            pltpu.VMEM((QE, HEADS * 64), f32),
            pltpu.VMEM((QE, HEADS * 64), f32),
            pltpu.VMEM((CHUNK, 128), f32),
            pltpu.SemaphoreType.DMA,
            pltpu.SemaphoreType.DMA,
            pltpu.SemaphoreType.DMA,
        ],
    )
    def gat_b_kernel(src_hbm, dst_hbm, ex_hbm, rdt_hbm, xwg_hbm, zeros_hbm,
                     out_hbm, acc_sh, dstidx, didx, exv, rdv, alv,
                     sidx0, sidx1, rows0, rows1, outc, sem, semr0, semr1):
        c = lax.axis_index("c")
        s = lax.axis_index("s")
        w = c * NS + s
        pltpu.sync_copy(zeros_hbm, acc_sh.at[pl.ds(s * RPT, RPT)])

        def zrow(e, carry):
            for j in range(4):
                outc[e, pl.ds(64 + j * 16, 16)] = jnp.zeros((16,), f32)
            return carry

        lax.fori_loop(0, CHUNK, zrow, 0)
        plsc.subcore_barrier()

        sidx = (sidx0, sidx1)
        rows = (rows0, rows1)
        semr = (semr0, semr1)

        def body(k, carry):
            ck = w * cpw + k
            base = ck * CHUNK
            pltpu.sync_copy(src_hbm.at[pl.ds(base, QE)], sidx0)
            pltpu.async_copy(xwg_hbm.at[sidx0], rows0, semr0)
            pltpu.sync_copy(dst_hbm.at[pl.ds(base, CHUNK)], dstidx)
            for h in range(HEADS):
                for j in range(CHUNK // 16):
                    sl = pl.ds(j * 16, 16)
                    didx[h, sl] = dstidx[sl] + (h * N_PAD)
            pltpu.sync_copy(ex_hbm.at[pl.ds(HEADS * ck, HEADS)], exv)
            for h in range(HEADS):
                pltpu.async_copy(rdt_hbm.at[didx.at[h]], rdv.at[h], sem)
            for h in range(HEADS):
                pltpu.make_async_copy(rdt_hbm.at[didx.at[h]], rdv.at[h], sem).wait()
            for h in range(HEADS):
                for j in range(CHUNK // 16):
                    sl = pl.ds(j * 16, 16)
                    alv[pl.ds(h * CHUNK + j * 16, 16)] = (
                        exv[h, sl] * rdv[h, sl] * 0.25
                    )
            for q in range(4):
                b = q % 2
                nb = 1 - b
                pltpu.make_async_copy(
                    xwg_hbm.at[sidx[b]], rows[b], semr[b]).wait()
                if q < 3:
                    pltpu.sync_copy(
                        src_hbm.at[pl.ds(base + (q + 1) * QE, QE)], sidx[nb])
                    pltpu.async_copy(xwg_hbm.at[sidx[nb]], rows[nb], semr[nb])

                def edge4(g, carry2, q=q, b=b):
                    go = g * 4
                    eo0 = go + q * QE
                    e16 = (eo0 // 16) * 16
                    lane0 = eo0 - e16
                    vv = [alv[pl.ds(h * CHUNK + e16, 16)] for h in range(HEADS)]
                    for sub in range(4):
                        lanev = jnp.full((16,), lane0 + sub, dtype=i32)
                        acc = [jnp.zeros((16,), f32) for _ in range(4)]
                        for h in range(HEADS):
                            ah = lax.gather(
                                vv[h], lanev[:, None],
                                lax.GatherDimensionNumbers(
                                    offset_dims=(), collapsed_slice_dims=(0,),
                                    start_index_map=(0,)),
                                (1,),
                                mode=lax.GatherScatterMode.PROMISE_IN_BOUNDS,
                            )
                            for j in range(4):
                                acc[j] = acc[j] + ah * rows[b][go + sub, pl.ds(h * 64 + j * 16, 16)]
                        for j in range(4):
                            outc[eo0 + sub, pl.ds(j * 16, 16)] = acc[j]
                    return carry2

                lax.fori_loop(0, QE // 4, edge4, 0)
            pltpu.sync_copy(outc, acc_sh.at[dstidx], add=True)
            return carry

        lax.fori_loop(0, cpw, body, 0)
        plsc.subcore_barrier()
        pltpu.sync_copy(
            acc_sh.at[pl.ds(s * RPT, RPT)], out_hbm.at[c, pl.ds(s * RPT, RPT)]
        )

    return gat_b_kernel


def _row_spec(dd):
    return pl.BlockSpec((BLK, dd), lambda i: (i, 0))


def _rep_spec(r, dd):
    return pl.BlockSpec((r, dd), lambda i: (0, 0))


def _t0_body(x_ref, w_ref, d0_ref, d1_ref, dinv_ref, xs_ref):
    dinv = lax.rsqrt(d0_ref[...] + d1_ref[...])
    dinv_ref[...] = dinv
    xs_ref[...] = jnp.dot(x_ref[...], w_ref[...], preferred_element_type=f32) * dinv


def _t0(x, w0, d0, d1):
    return pl.pallas_call(
        _t0_body,
        grid=(N // BLK,),
        in_specs=[_row_spec(128), _rep_spec(128, 128), _row_spec(1), _row_spec(1)],
        out_specs=[_row_spec(1), _row_spec(128)],
        out_shape=[
            jax.ShapeDtypeStruct((N, 1), f32),
            jax.ShapeDtypeStruct((N, 128), f32),
        ],
    )(x, w0, d0, d1)


def _t1_body(p0, p1, dinv, b0, w1, h1_ref, xs_ref):
    dv = dinv[...]
    h1 = jnp.maximum((p0[...] + p1[...]) * dv + b0[...], 0.0)
    h1_ref[...] = h1
    xs_ref[...] = jnp.dot(h1, w1[...], preferred_element_type=f32) * dv


def _t1(p0, p1, dinv, b0, w1):
    return pl.pallas_call(
        _t1_body,
        grid=(N // BLK,),
        in_specs=[_row_spec(128), _row_spec(128), _row_spec(1),
                  _rep_spec(1, 128), _rep_spec(128, 128)],
        out_specs=[_row_spec(128), _row_spec(128)],
        out_shape=[
            jax.ShapeDtypeStruct((N, 128), f32),
            jax.ShapeDtypeStruct((N, 128), f32),
        ],
    )(p0, p1, dinv, b0, w1)


def _t2_body(p0, p1, dinv, b1, h1, w2, xs_ref):
    dv = dinv[...]
    h2 = h1[...] + jnp.maximum((p0[...] + p1[...]) * dv + b1[...], 0.0)
    xs_ref[...] = jnp.dot(h2, w2[...], preferred_element_type=f32) * dv


def _t2(p0, p1, dinv, b1, h1, w2):
    return pl.pallas_call(
        _t2_body,
        grid=(N // BLK,),
        in_specs=[_row_spec(128), _row_spec(128), _row_spec(1),
                  _rep_spec(1, 128), _row_spec(128), _rep_spec(128, 64)],
        out_specs=[_row_spec(64)],
        out_shape=[jax.ShapeDtypeStruct((N, 64), f32)],
    )(p0, p1, dinv, b1, h1, w2)[0]


def _t3_body(p0, p1, dinv, b2, wg, acomb, xwg_ref, aout_ref):
    dv = dinv[...]
    h3 = jnp.maximum((p0[...] + p1[...]) * dv + b2[...], 0.0)
    xwg = jnp.dot(h3, wg[...], preferred_element_type=f32)
    xwg_ref[...] = xwg
    aout_ref[...] = jnp.dot(xwg, acomb[...], preferred_element_type=f32)


def _t3(p0, p1, dinv, b2, wg, acomb):
    return pl.pallas_call(
        _t3_body,
        grid=(N // BLK,),
        in_specs=[_row_spec(64), _row_spec(64), _row_spec(1),
                  _rep_spec(1, 64), _rep_spec(64, 256), _rep_spec(256, 128)],
        out_specs=[_row_spec(256), _row_spec(128)],
        out_shape=[
            jax.ShapeDtypeStruct((N, 256), f32),
            jax.ShapeDtypeStruct((N, 128), f32),
        ],
    )(p0, p1, dinv, b2, wg, acomb)


def _t3b_body(asrc_ref, adst_ref, c_ref):
    amax = jnp.max(asrc_ref[...], axis=0, keepdims=True)
    x = amax + adst_ref[...]
    c_ref[...] = jnp.maximum(x, 0.0) + 0.2 * jnp.minimum(x, 0.0)


def _t3b(asrc, adst):
    return pl.pallas_call(
        _t3b_body,
        grid=(1,),
        in_specs=[_rep_spec(N, HEADS), _rep_spec(N, HEADS)],
        out_specs=[_rep_spec(N, HEADS)],
        out_shape=[jax.ShapeDtypeStruct((N, HEADS), f32)],
    )(asrc, adst)[0]


def _t4_body(d0_ref, d1_ref, r_ref):
    r_ref[...] = 1.0 / (d0_ref[...] + d1_ref[...] + 1e-16)


def _t4(d0, d1):
    r = HEADS * N_PAD // 128
    return pl.pallas_call(
        _t4_body,
        grid=(1,),
        in_specs=[_rep_spec(r, 128), _rep_spec(r, 128)],
        out_specs=[_rep_spec(r, 128)],
        out_shape=[jax.ShapeDtypeStruct((r, 128), f32)],
    )(d0, d1)[0]


def _t5_body(p0, p1, bg, wcomb, bcomb, h4_ref, ge_ref, bo_ref):
    h4 = p0[...] + p1[...] + bg[...]
    h4_ref[...] = h4

    @pl.when(pl.program_id(0) == 0)
    def _():
        ge_ref[...] = jnp.zeros_like(ge_ref)

    ge_ref[...] += jnp.sum(h4, axis=0, keepdims=True) * (1.0 / N)
    bo_ref[...] = jnp.dot(h4, wcomb[...], preferred_element_type=f32) + bcomb[...]


def _t5(p0, p1, bg, wcomb, bcomb):
    return pl.pallas_call(
        _t5_body,
        grid=(N // BLK,),
        in_specs=[_row_spec(64), _row_spec(64), _rep_spec(1, 64),
                  _rep_spec(64, 128), _rep_spec(1, 128)],
        out_specs=[_row_spec(64), _rep_spec(1, 64), _row_spec(128)],
        out_shape=[
            jax.ShapeDtypeStruct((N, 64), f32),
            jax.ShapeDtypeStruct((1, 64), f32),
            jax.ShapeDtypeStruct((N, 128), f32),
        ],
    )(p0, p1, bg, wcomb, bcomb)


def _pad_rows(t):
    return jnp.pad(t, ((0, N_PAD - N), (0, 0)))


def _flat_t(a):
    return jnp.pad(a.T, ((0, 0), (0, N_PAD - N))).reshape(-1)


def kernel(x, edge_index, W0, b0, W1, b1, W2, b2, Wg, att_src, att_dst, bg,
           Wb, bb, Wo, bo):
    e_in = edge_index.shape[1]
    etot = e_in + N
    cpw = -(-etot // (NW * CHUNK))
    cpw += cpw % 2
    epad = cpw * NW * CHUNK

    loop = jnp.arange(N, dtype=i32)
    fill = TRASH + (jnp.arange(epad - etot, dtype=i32) % (N_PAD - N))
    srcp = jnp.concatenate([edge_index[0], loop, fill])
    dstp = jnp.concatenate([edge_index[1], loop, fill])
    nch = epad // CHUNK
    pack = jnp.stack([srcp.reshape(nch, CHUNK), dstp.reshape(nch, CHUNK)],
                     axis=1).reshape(2 * nch, CHUNK)
    pack = jnp.concatenate([pack, jnp.zeros((2, CHUNK), i32)], axis=0)

    z1 = jnp.zeros((RPT,), f32)
    z128 = jnp.zeros((RPT, 128), f32)
    z64 = jnp.zeros((RPT, 64), f32)
    z4 = jnp.zeros((HEADS * RPT,), f32)

    deg = _make_deg(cpw)(dstp, z1)
    d0 = deg[0, :N].reshape(N, 1)
    d1 = deg[1, :N].reshape(N, 1)
    dinv, xw0s = _t0(x, W0, d0, d1)

    gcn128 = _make_gcn2(cpw)
    p = gcn128(pack, _pad_rows(xw0s), z128)
    h1, xw1s = _t1(p[0, :N], p[1, :N], dinv, b0.reshape(1, -1), W1)

    p = gcn128(pack, _pad_rows(xw1s), z128)
    xw2s = _t2(p[0, :N], p[1, :N], dinv, b1.reshape(1, -1), h1, W2)

    xw2s_wide = jnp.pad(xw2s, ((0, 0), (0, 64)))
    p = gcn128(pack, _pad_rows(xw2s_wide), z128)
    p = p[:, :, :64]

    acomb = jnp.zeros((256, 128), f32)
    for h in range(HEADS):
        acomb = acomb.at[h * 64:(h + 1) * 64, h].set(att_src[h])
        acomb = acomb.at[h * 64:(h + 1) * 64, HEADS + h].set(att_dst[h])
    xwg, aout = _t3(p[0, :N], p[1, :N], dinv, b2.reshape(1, -1), Wg, acomb)
    asrc = aout[:, :HEADS]
    adst = aout[:, HEADS:2 * HEADS]
    cvals = _t3b(asrc, adst)

    den, ex = _make_gat_a(epad, cpw)(
        srcp, dstp, _flat_t(asrc), _flat_t(adst), _flat_t(cvals), z4
    )
    r = HEADS * N_PAD // 128
    rden = _t4(den[0].reshape(r, 128), den[1].reshape(r, 128)).reshape(-1)

    pg = _make_gat_b(epad, cpw)(srcp, dstp, ex, rden, _pad_rows(xwg), z128)

    wcomb = jnp.zeros((64, 128), f32)
    wcomb = wcomb.at[:, :2].set(Wb)
    wcomb = wcomb.at[:, 2:8].set(Wo)
    bcomb = jnp.zeros((1, 128), f32)
    bcomb = bcomb.at[0, :2].set(bb)
    bcomb = bcomb.at[0, 2:8].set(bo)

    h4, ge, bo_full = _t5(pg[0, :N, :64], pg[1, :N, :64], bg.reshape(1, -1), wcomb, bcomb)
    return h4, ge, bo_full[:, :2], bo_full[:, 2:8]


# speedup vs baseline: 2.2813x; 1.0739x over previous
import functools

import jax
import jax.numpy as jnp
from jax import lax
from jax.experimental import pallas as pl
from jax.experimental.pallas import tpu as pltpu
from jax.experimental.pallas import tpu_sc as plsc

f32 = jnp.float32
i32 = jnp.int32

N = 10000
HEADS = 4
NC = 2
NS = 16
NW = NC * NS
CHUNK = 128
N_PAD = 10240
TRASH = N
RPT = N_PAD // NS
BLK = 1000


def _mesh():
    return plsc.VectorSubcoreMesh(
        core_axis_name="c", subcore_axis_name="s", num_cores=NC, num_subcores=NS
    )


@functools.lru_cache(maxsize=None)
def _make_deg(cpw):
    @functools.partial(
        pl.kernel,
        out_type=jax.ShapeDtypeStruct((NC, N_PAD), f32),
        mesh=_mesh(),
        scratch_types=[
            pltpu.VMEM_SHARED((N_PAD,), f32),
            pltpu.VMEM((CHUNK,), i32),
            pltpu.VMEM((CHUNK,), f32),
        ],
    )
    def deg_kernel(dst_hbm, zeros_hbm, out_hbm, acc_sh, dstidx, ones_v):
        c = lax.axis_index("c")
        s = lax.axis_index("s")
        w = c * NS + s
        for j in range(CHUNK // 16):
            ones_v[pl.ds(j * 16, 16)] = jnp.ones((16,), f32)
        pltpu.sync_copy(zeros_hbm, acc_sh.at[pl.ds(s * RPT, RPT)])
        plsc.subcore_barrier()

        def body(k, carry):
            base = (w * cpw + k) * CHUNK
            pltpu.sync_copy(dst_hbm.at[pl.ds(base, CHUNK)], dstidx)
            pltpu.sync_copy(ones_v, acc_sh.at[dstidx], add=True)
            return carry

        lax.fori_loop(0, cpw, body, 0)
        plsc.subcore_barrier()
        pltpu.sync_copy(
            acc_sh.at[pl.ds(s * RPT, RPT)], out_hbm.at[c, pl.ds(s * RPT, RPT)]
        )

    return deg_kernel


@functools.lru_cache(maxsize=None)
def _make_gcn2(cpw):
    @functools.partial(
        pl.kernel,
        out_type=jax.ShapeDtypeStruct((NC, N_PAD, 128), f32),
        mesh=_mesh(),
        scratch_types=[
            pltpu.VMEM_SHARED((N_PAD, 128), f32),
            pltpu.VMEM((2, CHUNK), i32),
            pltpu.VMEM((2, CHUNK), i32),
            pltpu.VMEM((CHUNK, 128), f32),
            pltpu.VMEM((CHUNK, 128), f32),
            pltpu.SemaphoreType.DMA,
            pltpu.SemaphoreType.DMA,
        ],
    )
    def gcn_kernel(pack_hbm, tab_hbm, zeros_hbm, out_hbm,
                   acc_sh, idxa, idxb, rowsa, rowsb, sema, semb):
        c = lax.axis_index("c")
        s = lax.axis_index("s")
        w = c * NS + s
        c0 = w * cpw
        pltpu.sync_copy(zeros_hbm, acc_sh.at[pl.ds(s * RPT, RPT)])
        plsc.subcore_barrier()
        pltpu.sync_copy(pack_hbm.at[pl.ds(2 * c0, 2)], idxa)
        pltpu.async_copy(tab_hbm.at[idxa.at[0]], rowsa, sema)

        def body(k2, carry):
            ck = c0 + 2 * k2
            pltpu.make_async_copy(tab_hbm.at[idxa.at[0]], rowsa, sema).wait()
            pltpu.sync_copy(pack_hbm.at[pl.ds(2 * (ck + 1), 2)], idxb)
            pltpu.async_copy(tab_hbm.at[idxb.at[0]], rowsb, semb)
            pltpu.sync_copy(rowsa, acc_sh.at[idxa.at[1]], add=True)
            pltpu.make_async_copy(tab_hbm.at[idxb.at[0]], rowsb, semb).wait()
            pltpu.sync_copy(pack_hbm.at[pl.ds(2 * (ck + 2), 2)], idxa)
            pltpu.async_copy(tab_hbm.at[idxa.at[0]], rowsa, sema)
            pltpu.sync_copy(rowsb, acc_sh.at[idxb.at[1]], add=True)
            return carry

        lax.fori_loop(0, cpw // 2, body, 0)
        pltpu.make_async_copy(tab_hbm.at[idxa.at[0]], rowsa, sema).wait()
        plsc.subcore_barrier()
        pltpu.sync_copy(
            acc_sh.at[pl.ds(s * RPT, RPT)], out_hbm.at[c, pl.ds(s * RPT, RPT)]
        )

    return gcn_kernel


@functools.lru_cache(maxsize=None)
def _make_gcn(d, cpw):
    @functools.partial(
        pl.kernel,
        out_type=jax.ShapeDtypeStruct((NC, N_PAD, d), f32),
        mesh=_mesh(),
        scratch_types=[
            pltpu.VMEM_SHARED((N_PAD, d), f32),
            pltpu.VMEM((CHUNK,), i32),
            pltpu.VMEM((CHUNK,), i32),
            pltpu.VMEM((CHUNK, d), f32),
            pltpu.SemaphoreType.DMA,
        ],
    )
    def gcn_kernel(src_hbm, dst_hbm, tab_hbm, zeros_hbm, out_hbm,
                   acc_sh, srcidx, dstidx, rows, sem):
        c = lax.axis_index("c")
        s = lax.axis_index("s")
        w = c * NS + s
        pltpu.sync_copy(zeros_hbm, acc_sh.at[pl.ds(s * RPT, RPT)])
        plsc.subcore_barrier()

        def body(k, carry):
            base = (w * cpw + k) * CHUNK
            pltpu.sync_copy(src_hbm.at[pl.ds(base, CHUNK)], srcidx)
            pltpu.sync_copy(dst_hbm.at[pl.ds(base, CHUNK)], dstidx)
            pltpu.async_copy(tab_hbm.at[srcidx], rows, sem).wait()
            pltpu.sync_copy(rows, acc_sh.at[dstidx], add=True)
            return carry

        lax.fori_loop(0, cpw, body, 0)
        plsc.subcore_barrier()
        pltpu.sync_copy(
            acc_sh.at[pl.ds(s * RPT, RPT)], out_hbm.at[c, pl.ds(s * RPT, RPT)]
        )

    return gcn_kernel


@functools.lru_cache(maxsize=None)
def _make_gat_a(epad, cpw):
    @functools.partial(
        pl.kernel,
        out_type=(
            jax.ShapeDtypeStruct((NC, HEADS * N_PAD), f32),
            jax.ShapeDtypeStruct((HEADS * (epad // CHUNK), CHUNK), f32),
        ),
        mesh=_mesh(),
        scratch_types=[
            pltpu.VMEM_SHARED((HEADS * N_PAD,), f32),
            pltpu.VMEM((CHUNK,), i32),
            pltpu.VMEM((CHUNK,), i32),
            pltpu.VMEM((HEADS, CHUNK), i32),
            pltpu.VMEM((HEADS, CHUNK), i32),
            pltpu.VMEM((HEADS, CHUNK), i32),
            pltpu.VMEM((HEADS, CHUNK), i32),
            pltpu.VMEM((HEADS, CHUNK), f32),
            pltpu.VMEM((HEADS, CHUNK), f32),
            pltpu.VMEM((HEADS, CHUNK), f32),
            pltpu.VMEM((HEADS, CHUNK), f32),
            pltpu.SemaphoreType.DMA,
            pltpu.SemaphoreType.DMA,
        ],
    )
    def gat_a_kernel(src_hbm, dst_hbm, at_hbm, dt_hbm, ct_hbm, zeros_hbm,
                     den_hbm, ex_hbm,
                     acc_sh, srcidx, dstidx, sidxa, didxa, sidxb, didxb,
                     av, dv, cv, exv, sem, sems):
        c = lax.axis_index("c")
        s = lax.axis_index("s")
        w = c * NS + s
        c0 = w * cpw
        pltpu.sync_copy(zeros_hbm, acc_sh.at[pl.ds(s * (HEADS * RPT), HEADS * RPT)])
        plsc.subcore_barrier()

        def build_and_fire(ck, sx, dx):
            base = ck * CHUNK
            pltpu.sync_copy(src_hbm.at[pl.ds(base, CHUNK)], srcidx)
            pltpu.sync_copy(dst_hbm.at[pl.ds(base, CHUNK)], dstidx)
            for h in range(HEADS):
                for j in range(CHUNK // 16):
                    sl = pl.ds(j * 16, 16)
                    sx[h, sl] = srcidx[sl] + (h * N_PAD)
                    dx[h, sl] = dstidx[sl] + (h * N_PAD)
            for h in range(HEADS):
                pltpu.async_copy(at_hbm.at[sx.at[h]], av.at[h], sem)
                pltpu.async_copy(dt_hbm.at[dx.at[h]], dv.at[h], sem)
                pltpu.async_copy(ct_hbm.at[dx.at[h]], cv.at[h], sem)

        def consume(ck, sx, dx, sx_n, dx_n):
            for h in range(HEADS):
                pltpu.make_async_copy(at_hbm.at[sx.at[h]], av.at[h], sem).wait()
                pltpu.make_async_copy(dt_hbm.at[dx.at[h]], dv.at[h], sem).wait()
                pltpu.make_async_copy(ct_hbm.at[dx.at[h]], cv.at[h], sem).wait()
            for h in range(HEADS):
                for j in range(CHUNK // 16):
                    sl = pl.ds(j * 16, 16)
                    x = av[h, sl] + dv[h, sl]
                    e = jnp.maximum(x, 0.0) + 0.2 * jnp.minimum(x, 0.0)
                    exv[h, sl] = jnp.exp(e - cv[h, sl])
            build_and_fire(ck + 1, sx_n, dx_n)
            pltpu.sync_copy(exv, ex_hbm.at[pl.ds(HEADS * ck, HEADS)])
            for h in range(HEADS):
                pltpu.async_copy(exv.at[h], acc_sh.at[dx.at[h]], sems, add=True)
            for h in range(HEADS):
                pltpu.make_async_copy(exv.at[h], acc_sh.at[dx.at[h]], sems).wait()

        build_and_fire(c0, sidxa, didxa)

        def body(k2, carry):
            ck = c0 + 2 * k2
            consume(ck, sidxa, didxa, sidxb, didxb)
            consume(ck + 1, sidxb, didxb, sidxa, didxa)
            return carry

        lax.fori_loop(0, cpw // 2, body, 0)
        for h in range(HEADS):
            pltpu.make_async_copy(at_hbm.at[sidxa.at[h]], av.at[h], sem).wait()
            pltpu.make_async_copy(dt_hbm.at[didxa.at[h]], dv.at[h], sem).wait()
            pltpu.make_async_copy(ct_hbm.at[didxa.at[h]], cv.at[h], sem).wait()
        plsc.subcore_barrier()
        pltpu.sync_copy(
            acc_sh.at[pl.ds(s * (HEADS * RPT), HEADS * RPT)],
            den_hbm.at[c, pl.ds(s * (HEADS * RPT), HEADS * RPT)],
        )

    return gat_a_kernel


@functools.lru_cache(maxsize=None)
def _make_gat_b(epad, cpw):
    QE = CHUNK // 4

    @functools.partial(
        pl.kernel,
        out_type=jax.ShapeDtypeStruct((NC, N_PAD, 128), f32),
        mesh=_mesh(),
        scratch_types=[
            pltpu.VMEM_SHARED((N_PAD, 128), f32),
            pltpu.VMEM((CHUNK,), i32),
            pltpu.VMEM((CHUNK,), i32),
            pltpu.VMEM((HEADS, CHUNK), i32),
            pltpu.VMEM((HEADS, CHUNK), f32),
            pltpu.VMEM((HEADS, CHUNK), f32),
            pltpu.VMEM((HEADS * CHUNK,), f32),
            pltpu.VMEM((QE,), i32),
            pltpu.VMEM((QE,), i32),
            pltpu.VMEM((QE, HEADS * 64), f32),
            pltpu.VMEM((QE, HEADS * 64), f32),
            pltpu.VMEM((CHUNK, 128), f32),
            pltpu.SemaphoreType.DMA,
            pltpu.SemaphoreType.DMA,
            pltpu.SemaphoreType.DMA,
            pltpu.SemaphoreType.DMA,
        ],
    )
    def gat_b_kernel(src_hbm, dst_hbm, ex_hbm, rdt_hbm, xwg_hbm, zeros_hbm,
                     out_hbm, acc_sh, dstidx, dstidx2, didx, exv, rdv, alv,
                     sidx0, sidx1, rows0, rows1, outc, sem, seme, semr0, semr1):
        c = lax.axis_index("c")
        s = lax.axis_index("s")
        w = c * NS + s
        pltpu.sync_copy(zeros_hbm, acc_sh.at[pl.ds(s * RPT, RPT)])

        def zrow(e, carry):
            for j in range(4):
                outc[e, pl.ds(64 + j * 16, 16)] = jnp.zeros((16,), f32)
            return carry

        lax.fori_loop(0, CHUNK, zrow, 0)
        plsc.subcore_barrier()

        sidx = (sidx0, sidx1)
        rows = (rows0, rows1)
        semr = (semr0, semr1)

        def fire_head(ck, dx):
            pltpu.sync_copy(dst_hbm.at[pl.ds(ck * CHUNK, CHUNK)], dx)
            for h in range(HEADS):
                for j in range(CHUNK // 16):
                    sl = pl.ds(j * 16, 16)
                    didx[h, sl] = dx[sl] + (h * N_PAD)
            pltpu.async_copy(ex_hbm.at[pl.ds(HEADS * ck, HEADS)], exv, seme)
            for h in range(HEADS):
                pltpu.async_copy(rdt_hbm.at[didx.at[h]], rdv.at[h], sem)

        def chunk(ck, dx, dx_n, carry):
            base = ck * CHUNK
            pltpu.sync_copy(src_hbm.at[pl.ds(base, QE)], sidx0)
            pltpu.async_copy(xwg_hbm.at[sidx0], rows0, semr0)
            pltpu.make_async_copy(
                ex_hbm.at[pl.ds(HEADS * ck, HEADS)], exv, seme).wait()
            for h in range(HEADS):
                pltpu.make_async_copy(rdt_hbm.at[didx.at[h]], rdv.at[h], sem).wait()
            for h in range(HEADS):
                for j in range(CHUNK // 16):
                    sl = pl.ds(j * 16, 16)
                    alv[pl.ds(h * CHUNK + j * 16, 16)] = (
                        exv[h, sl] * rdv[h, sl] * 0.25
                    )
            fire_head(ck + 1, dx_n)
            for q in range(4):
                b = q % 2
                nb = 1 - b
                pltpu.make_async_copy(
                    xwg_hbm.at[sidx[b]], rows[b], semr[b]).wait()
                if q < 3:
                    pltpu.sync_copy(
                        src_hbm.at[pl.ds(base + (q + 1) * QE, QE)], sidx[nb])
                    pltpu.async_copy(xwg_hbm.at[sidx[nb]], rows[nb], semr[nb])

                def edge4(g, carry2, q=q, b=b):
                    go = g * 4
                    eo0 = go + q * QE
                    e16 = (eo0 // 16) * 16
                    lane0 = eo0 - e16
                    vv = [alv[pl.ds(h * CHUNK + e16, 16)] for h in range(HEADS)]
                    for sub in range(4):
                        lanev = jnp.full((16,), lane0 + sub, dtype=i32)
                        acc = [jnp.zeros((16,), f32) for _ in range(4)]
                        for h in range(HEADS):
                            ah = lax.gather(
                                vv[h], lanev[:, None],
                                lax.GatherDimensionNumbers(
                                    offset_dims=(), collapsed_slice_dims=(0,),
                                    start_index_map=(0,)),
                                (1,),
                                mode=lax.GatherScatterMode.PROMISE_IN_BOUNDS,
                            )
                            for j in range(4):
                                acc[j] = acc[j] + ah * rows[b][go + sub, pl.ds(h * 64 + j * 16, 16)]
                        for j in range(4):
                            outc[eo0 + sub, pl.ds(j * 16, 16)] = acc[j]
                    return carry2

                lax.fori_loop(0, QE // 4, edge4, 0)
            pltpu.sync_copy(outc, acc_sh.at[dx], add=True)
            return carry

        c0 = w * cpw
        fire_head(c0, dstidx)

        def body(k2, carry):
            ck = c0 + 2 * k2
            carry = chunk(ck, dstidx, dstidx2, carry)
            carry = chunk(ck + 1, dstidx2, dstidx, carry)
            return carry

        lax.fori_loop(0, cpw // 2, body, 0)
        pltpu.make_async_copy(
            ex_hbm.at[pl.ds(0, HEADS)], exv, seme).wait()
        for h in range(HEADS):
            pltpu.make_async_copy(rdt_hbm.at[didx.at[h]], rdv.at[h], sem).wait()
        plsc.subcore_barrier()
        pltpu.sync_copy(
            acc_sh.at[pl.ds(s * RPT, RPT)], out_hbm.at[c, pl.ds(s * RPT, RPT)]
        )

    return gat_b_kernel


def _row_spec(dd):
    return pl.BlockSpec((BLK, dd), lambda i: (i, 0))


def _rep_spec(r, dd):
    return pl.BlockSpec((r, dd), lambda i: (0, 0))


def _t0_body(x_ref, w_ref, d0_ref, d1_ref, dinv_ref, xs_ref):
    dinv = lax.rsqrt(d0_ref[...] + d1_ref[...])
    dinv_ref[...] = dinv
    xs_ref[...] = jnp.dot(x_ref[...], w_ref[...], preferred_element_type=f32) * dinv


def _t0(x, w0, d0, d1):
    return pl.pallas_call(
        _t0_body,
        grid=(N // BLK,),
        in_specs=[_row_spec(128), _rep_spec(128, 128), _row_spec(1), _row_spec(1)],
        out_specs=[_row_spec(1), _row_spec(128)],
        out_shape=[
            jax.ShapeDtypeStruct((N, 1), f32),
            jax.ShapeDtypeStruct((N, 128), f32),
        ],
    )(x, w0, d0, d1)


def _t1_body(p0, p1, dinv, b0, w1, h1_ref, xs_ref):
    dv = dinv[...]
    h1 = jnp.maximum((p0[...] + p1[...]) * dv + b0[...], 0.0)
    h1_ref[...] = h1
    xs_ref[...] = jnp.dot(h1, w1[...], preferred_element_type=f32) * dv


def _t1(p0, p1, dinv, b0, w1):
    return pl.pallas_call(
        _t1_body,
        grid=(N // BLK,),
        in_specs=[_row_spec(128), _row_spec(128), _row_spec(1),
                  _rep_spec(1, 128), _rep_spec(128, 128)],
        out_specs=[_row_spec(128), _row_spec(128)],
        out_shape=[
            jax.ShapeDtypeStruct((N, 128), f32),
            jax.ShapeDtypeStruct((N, 128), f32),
        ],
    )(p0, p1, dinv, b0, w1)


def _t2_body(p0, p1, dinv, b1, h1, w2, xs_ref):
    dv = dinv[...]
    h2 = h1[...] + jnp.maximum((p0[...] + p1[...]) * dv + b1[...], 0.0)
    xs_ref[...] = jnp.dot(h2, w2[...], preferred_element_type=f32) * dv


def _t2(p0, p1, dinv, b1, h1, w2):
    return pl.pallas_call(
        _t2_body,
        grid=(N // BLK,),
        in_specs=[_row_spec(128), _row_spec(128), _row_spec(1),
                  _rep_spec(1, 128), _row_spec(128), _rep_spec(128, 64)],
        out_specs=[_row_spec(64)],
        out_shape=[jax.ShapeDtypeStruct((N, 64), f32)],
    )(p0, p1, dinv, b1, h1, w2)[0]


def _t3_body(p0, p1, dinv, b2, wg, acomb, xwg_ref, aout_ref):
    dv = dinv[...]
    h3 = jnp.maximum((p0[...] + p1[...]) * dv + b2[...], 0.0)
    xwg = jnp.dot(h3, wg[...], preferred_element_type=f32)
    xwg_ref[...] = xwg
    aout_ref[...] = jnp.dot(xwg, acomb[...], preferred_element_type=f32)


def _t3(p0, p1, dinv, b2, wg, acomb):
    return pl.pallas_call(
        _t3_body,
        grid=(N // BLK,),
        in_specs=[_row_spec(64), _row_spec(64), _row_spec(1),
                  _rep_spec(1, 64), _rep_spec(64, 256), _rep_spec(256, 128)],
        out_specs=[_row_spec(256), _row_spec(128)],
        out_shape=[
            jax.ShapeDtypeStruct((N, 256), f32),
            jax.ShapeDtypeStruct((N, 128), f32),
        ],
    )(p0, p1, dinv, b2, wg, acomb)


def _t3b_body(asrc_ref, adst_ref, c_ref):
    amax = jnp.max(asrc_ref[...], axis=0, keepdims=True)
    x = amax + adst_ref[...]
    c_ref[...] = jnp.maximum(x, 0.0) + 0.2 * jnp.minimum(x, 0.0)


def _t3b(asrc, adst):
    return pl.pallas_call(
        _t3b_body,
        grid=(1,),
        in_specs=[_rep_spec(N, HEADS), _rep_spec(N, HEADS)],
        out_specs=[_rep_spec(N, HEADS)],
        out_shape=[jax.ShapeDtypeStruct((N, HEADS), f32)],
    )(asrc, adst)[0]


def _t4_body(d0_ref, d1_ref, r_ref):
    r_ref[...] = 1.0 / (d0_ref[...] + d1_ref[...] + 1e-16)


def _t4(d0, d1):
    r = HEADS * N_PAD // 128
    return pl.pallas_call(
        _t4_body,
        grid=(1,),
        in_specs=[_rep_spec(r, 128), _rep_spec(r, 128)],
        out_specs=[_rep_spec(r, 128)],
        out_shape=[jax.ShapeDtypeStruct((r, 128), f32)],
    )(d0, d1)[0]


def _t5_body(p0, p1, bg, wcomb, bcomb, h4_ref, ge_ref, bo_ref):
    h4 = p0[...] + p1[...] + bg[...]
    h4_ref[...] = h4

    @pl.when(pl.program_id(0) == 0)
    def _():
        ge_ref[...] = jnp.zeros_like(ge_ref)

    ge_ref[...] += jnp.sum(h4, axis=0, keepdims=True) * (1.0 / N)
    bo_ref[...] = jnp.dot(h4, wcomb[...], preferred_element_type=f32) + bcomb[...]


def _t5(p0, p1, bg, wcomb, bcomb):
    return pl.pallas_call(
        _t5_body,
        grid=(N // BLK,),
        in_specs=[_row_spec(64), _row_spec(64), _rep_spec(1, 64),
                  _rep_spec(64, 128), _rep_spec(1, 128)],
        out_specs=[_row_spec(64), _rep_spec(1, 64), _row_spec(128)],
        out_shape=[
            jax.ShapeDtypeStruct((N, 64), f32),
            jax.ShapeDtypeStruct((1, 64), f32),
            jax.ShapeDtypeStruct((N, 128), f32),
        ],
    )(p0, p1, bg, wcomb, bcomb)


def _pad_rows(t):
    return jnp.pad(t, ((0, N_PAD - N), (0, 0)))


def _flat_t(a):
    return jnp.pad(a.T, ((0, 0), (0, N_PAD - N))).reshape(-1)


def kernel(x, edge_index, W0, b0, W1, b1, W2, b2, Wg, att_src, att_dst, bg,
           Wb, bb, Wo, bo):
    e_in = edge_index.shape[1]
    etot = e_in + N
    cpw = -(-etot // (NW * CHUNK))
    cpw += cpw % 2
    epad = cpw * NW * CHUNK

    loop = jnp.arange(N, dtype=i32)
    fill = TRASH + (jnp.arange(epad - etot, dtype=i32) % (N_PAD - N))
    zc = jnp.zeros((CHUNK,), i32)
    srcp = jnp.concatenate([edge_index[0], loop, fill, zc])
    dstp = jnp.concatenate([edge_index[1], loop, fill, zc])
    nch = epad // CHUNK
    pack = jnp.stack([srcp[:epad].reshape(nch, CHUNK),
                      dstp[:epad].reshape(nch, CHUNK)],
                     axis=1).reshape(2 * nch, CHUNK)
    pack = jnp.concatenate([pack, jnp.zeros((2, CHUNK), i32)], axis=0)

    z1 = jnp.zeros((RPT,), f32)
    z128 = jnp.zeros((RPT, 128), f32)
    z64 = jnp.zeros((RPT, 64), f32)
    z4 = jnp.zeros((HEADS * RPT,), f32)

    deg = _make_deg(cpw)(dstp, z1)
    d0 = deg[0, :N].reshape(N, 1)
    d1 = deg[1, :N].reshape(N, 1)
    dinv, xw0s = _t0(x, W0, d0, d1)

    gcn128 = _make_gcn2(cpw)
    p = gcn128(pack, _pad_rows(xw0s), z128)
    h1, xw1s = _t1(p[0, :N], p[1, :N], dinv, b0.reshape(1, -1), W1)

    p = gcn128(pack, _pad_rows(xw1s), z128)
    xw2s = _t2(p[0, :N], p[1, :N], dinv, b1.reshape(1, -1), h1, W2)

    xw2s_wide = jnp.pad(xw2s, ((0, 0), (0, 64)))
    p = gcn128(pack, _pad_rows(xw2s_wide), z128)
    p = p[:, :, :64]

    acomb = jnp.zeros((256, 128), f32)
    for h in range(HEADS):
        acomb = acomb.at[h * 64:(h + 1) * 64, h].set(att_src[h])
        acomb = acomb.at[h * 64:(h + 1) * 64, HEADS + h].set(att_dst[h])
    xwg, aout = _t3(p[0, :N], p[1, :N], dinv, b2.reshape(1, -1), Wg, acomb)
    asrc = aout[:, :HEADS]
    adst = aout[:, HEADS:2 * HEADS]
    cvals = _t3b(asrc, adst)

    den, ex = _make_gat_a(epad, cpw)(
        srcp, dstp, _flat_t(asrc), _flat_t(adst), _flat_t(cvals), z4
    )
    ex = jnp.pad(ex, ((0, HEADS), (0, 0)))
    r = HEADS * N_PAD // 128
    rden = _t4(den[0].reshape(r, 128), den[1].reshape(r, 128)).reshape(-1)

    pg = _make_gat_b(epad, cpw)(srcp, dstp, ex, rden, _pad_rows(xwg), z128)

    wcomb = jnp.zeros((64, 128), f32)
    wcomb = wcomb.at[:, :2].set(Wb)
    wcomb = wcomb.at[:, 2:8].set(Wo)
    bcomb = jnp.zeros((1, 128), f32)
    bcomb = bcomb.at[0, :2].set(bb)
    bcomb = bcomb.at[0, 2:8].set(bo)

    h4, ge, bo_full = _t5(pg[0, :N, :64], pg[1, :N, :64], bg.reshape(1, -1), wcomb, bcomb)
    return h4, ge, bo_full[:, :2], bo_full[:, 2:8]


# speedup vs baseline: 2.2816x; 1.0001x over previous
import functools

import jax
import jax.numpy as jnp
from jax import lax
from jax.experimental import pallas as pl
from jax.experimental.pallas import tpu as pltpu
from jax.experimental.pallas import tpu_sc as plsc

f32 = jnp.float32
i32 = jnp.int32

N = 10000
HEADS = 4
NC = 2
NS = 16
NW = NC * NS
CHUNK = 128
N_PAD = 10240
TRASH = N
RPT = N_PAD // NS
BLK = 1000


def _mesh():
    return plsc.VectorSubcoreMesh(
        core_axis_name="c", subcore_axis_name="s", num_cores=NC, num_subcores=NS
    )


@functools.lru_cache(maxsize=None)
def _make_deg(cpw):
    @functools.partial(
        pl.kernel,
        out_type=jax.ShapeDtypeStruct((NC, N_PAD), f32),
        mesh=_mesh(),
        scratch_types=[
            pltpu.VMEM_SHARED((N_PAD,), f32),
            pltpu.VMEM((CHUNK,), i32),
            pltpu.VMEM((CHUNK,), f32),
        ],
    )
    def deg_kernel(dst_hbm, zeros_hbm, out_hbm, acc_sh, dstidx, ones_v):
        c = lax.axis_index("c")
        s = lax.axis_index("s")
        w = c * NS + s
        for j in range(CHUNK // 16):
            ones_v[pl.ds(j * 16, 16)] = jnp.ones((16,), f32)
        pltpu.sync_copy(zeros_hbm, acc_sh.at[pl.ds(s * RPT, RPT)])
        plsc.subcore_barrier()

        def body(k, carry):
            base = (w * cpw + k) * CHUNK
            pltpu.sync_copy(dst_hbm.at[pl.ds(base, CHUNK)], dstidx)
            pltpu.sync_copy(ones_v, acc_sh.at[dstidx], add=True)
            return carry

        lax.fori_loop(0, cpw, body, 0)
        plsc.subcore_barrier()
        pltpu.sync_copy(
            acc_sh.at[pl.ds(s * RPT, RPT)], out_hbm.at[c, pl.ds(s * RPT, RPT)]
        )

    return deg_kernel


@functools.lru_cache(maxsize=None)
def _make_gcn2(cpw):
    @functools.partial(
        pl.kernel,
        out_type=jax.ShapeDtypeStruct((NC, N_PAD, 128), f32),
        mesh=_mesh(),
        scratch_types=[
            pltpu.VMEM_SHARED((N_PAD, 128), f32),
            pltpu.VMEM((2, CHUNK), i32),
            pltpu.VMEM((2, CHUNK), i32),
            pltpu.VMEM((CHUNK, 128), f32),
            pltpu.VMEM((CHUNK, 128), f32),
            pltpu.SemaphoreType.DMA,
            pltpu.SemaphoreType.DMA,
        ],
    )
    def gcn_kernel(pack_hbm, tab_hbm, zeros_hbm, out_hbm,
                   acc_sh, idxa, idxb, rowsa, rowsb, sema, semb):
        c = lax.axis_index("c")
        s = lax.axis_index("s")
        w = c * NS + s
        c0 = w * cpw
        pltpu.sync_copy(zeros_hbm, acc_sh.at[pl.ds(s * RPT, RPT)])
        plsc.subcore_barrier()
        pltpu.sync_copy(pack_hbm.at[pl.ds(2 * c0, 2)], idxa)
        pltpu.async_copy(tab_hbm.at[idxa.at[0]], rowsa, sema)

        def body(k2, carry):
            ck = c0 + 2 * k2
            pltpu.make_async_copy(tab_hbm.at[idxa.at[0]], rowsa, sema).wait()
            pltpu.sync_copy(pack_hbm.at[pl.ds(2 * (ck + 1), 2)], idxb)
            pltpu.async_copy(tab_hbm.at[idxb.at[0]], rowsb, semb)
            pltpu.sync_copy(rowsa, acc_sh.at[idxa.at[1]], add=True)
            pltpu.make_async_copy(tab_hbm.at[idxb.at[0]], rowsb, semb).wait()
            pltpu.sync_copy(pack_hbm.at[pl.ds(2 * (ck + 2), 2)], idxa)
            pltpu.async_copy(tab_hbm.at[idxa.at[0]], rowsa, sema)
            pltpu.sync_copy(rowsb, acc_sh.at[idxb.at[1]], add=True)
            return carry

        lax.fori_loop(0, cpw // 2, body, 0)
        pltpu.make_async_copy(tab_hbm.at[idxa.at[0]], rowsa, sema).wait()
        plsc.subcore_barrier()
        pltpu.sync_copy(
            acc_sh.at[pl.ds(s * RPT, RPT)], out_hbm.at[c, pl.ds(s * RPT, RPT)]
        )

    return gcn_kernel


@functools.lru_cache(maxsize=None)
def _make_gat_a(epad, cpw):
    @functools.partial(
        pl.kernel,
        out_type=(
            jax.ShapeDtypeStruct((NC, HEADS * N_PAD), f32),
            jax.ShapeDtypeStruct((HEADS * (epad // CHUNK), CHUNK), f32),
        ),
        mesh=_mesh(),
        scratch_types=[
            pltpu.VMEM_SHARED((HEADS * N_PAD,), f32),
            pltpu.VMEM((CHUNK,), i32),
            pltpu.VMEM((CHUNK,), i32),
            pltpu.VMEM((HEADS, CHUNK), i32),
            pltpu.VMEM((HEADS, CHUNK), i32),
            pltpu.VMEM((HEADS, CHUNK), i32),
            pltpu.VMEM((HEADS, CHUNK), i32),
            pltpu.VMEM((HEADS, CHUNK), f32),
            pltpu.VMEM((HEADS, CHUNK), f32),
            pltpu.VMEM((HEADS, CHUNK), f32),
            pltpu.VMEM((HEADS, CHUNK), f32),
            pltpu.SemaphoreType.DMA,
            pltpu.SemaphoreType.DMA,
        ],
    )
    def gat_a_kernel(src_hbm, dst_hbm, at_hbm, dt_hbm, ct_hbm, zeros_hbm,
                     den_hbm, ex_hbm,
                     acc_sh, srcidx, dstidx, sidxa, didxa, sidxb, didxb,
                     av, dv, cv, exv, sem, sems):
        c = lax.axis_index("c")
        s = lax.axis_index("s")
        w = c * NS + s
        c0 = w * cpw
        pltpu.sync_copy(zeros_hbm, acc_sh.at[pl.ds(s * (HEADS * RPT), HEADS * RPT)])
        plsc.subcore_barrier()

        def build_and_fire(ck, sx, dx):
            base = ck * CHUNK
            pltpu.sync_copy(src_hbm.at[pl.ds(base, CHUNK)], srcidx)
            pltpu.sync_copy(dst_hbm.at[pl.ds(base, CHUNK)], dstidx)
            for h in range(HEADS):
                for j in range(CHUNK // 16):
                    sl = pl.ds(j * 16, 16)
                    sx[h, sl] = srcidx[sl] + (h * N_PAD)
                    dx[h, sl] = dstidx[sl] + (h * N_PAD)
            for h in range(HEADS):
                pltpu.async_copy(at_hbm.at[sx.at[h]], av.at[h], sem)
                pltpu.async_copy(dt_hbm.at[dx.at[h]], dv.at[h], sem)
                pltpu.async_copy(ct_hbm.at[dx.at[h]], cv.at[h], sem)

        def consume(ck, sx, dx, sx_n, dx_n):
            for h in range(HEADS):
                pltpu.make_async_copy(at_hbm.at[sx.at[h]], av.at[h], sem).wait()
                pltpu.make_async_copy(dt_hbm.at[dx.at[h]], dv.at[h], sem).wait()
                pltpu.make_async_copy(ct_hbm.at[dx.at[h]], cv.at[h], sem).wait()
            for h in range(HEADS):
                for j in range(CHUNK // 16):
                    sl = pl.ds(j * 16, 16)
                    x = av[h, sl] + dv[h, sl]
                    e = jnp.maximum(x, 0.0) + 0.2 * jnp.minimum(x, 0.0)
                    exv[h, sl] = jnp.exp(e - cv[h, sl])
            build_and_fire(ck + 1, sx_n, dx_n)
            pltpu.sync_copy(exv, ex_hbm.at[pl.ds(HEADS * ck, HEADS)])
            for h in range(HEADS):
                pltpu.async_copy(exv.at[h], acc_sh.at[dx.at[h]], sems, add=True)
            for h in range(HEADS):
                pltpu.make_async_copy(exv.at[h], acc_sh.at[dx.at[h]], sems).wait()

        build_and_fire(c0, sidxa, didxa)

        def body(k2, carry):
            ck = c0 + 2 * k2
            consume(ck, sidxa, didxa, sidxb, didxb)
            consume(ck + 1, sidxb, didxb, sidxa, didxa)
            return carry

        lax.fori_loop(0, cpw // 2, body, 0)
        for h in range(HEADS):
            pltpu.make_async_copy(at_hbm.at[sidxa.at[h]], av.at[h], sem).wait()
            pltpu.make_async_copy(dt_hbm.at[didxa.at[h]], dv.at[h], sem).wait()
            pltpu.make_async_copy(ct_hbm.at[didxa.at[h]], cv.at[h], sem).wait()
        plsc.subcore_barrier()
        pltpu.sync_copy(
            acc_sh.at[pl.ds(s * (HEADS * RPT), HEADS * RPT)],
            den_hbm.at[c, pl.ds(s * (HEADS * RPT), HEADS * RPT)],
        )

    return gat_a_kernel


@functools.lru_cache(maxsize=None)
def _make_gat_b(epad, cpw):
    QE = CHUNK // 4

    @functools.partial(
        pl.kernel,
        out_type=jax.ShapeDtypeStruct((NC, N_PAD, 128), f32),
        mesh=_mesh(),
        scratch_types=[
            pltpu.VMEM_SHARED((N_PAD, 128), f32),
            pltpu.VMEM((CHUNK,), i32),
            pltpu.VMEM((CHUNK,), i32),
            pltpu.VMEM((HEADS, CHUNK), i32),
            pltpu.VMEM((HEADS, CHUNK), f32),
            pltpu.VMEM((HEADS, CHUNK), f32),
            pltpu.VMEM((HEADS * CHUNK,), f32),
            pltpu.VMEM((QE,), i32),
            pltpu.VMEM((QE,), i32),
            pltpu.VMEM((QE, HEADS * 64), f32),
            pltpu.VMEM((QE, HEADS * 64), f32),
            pltpu.VMEM((CHUNK, 128), f32),
            pltpu.SemaphoreType.DMA,
            pltpu.SemaphoreType.DMA,
            pltpu.SemaphoreType.DMA,
            pltpu.SemaphoreType.DMA,
        ],
    )
    def gat_b_kernel(src_hbm, dst_hbm, ex_hbm, rdt_hbm, xwg_hbm, zeros_hbm,
                     out_hbm, acc_sh, dstidx, dstidx2, didx, exv, rdv, alv,
                     sidx0, sidx1, rows0, rows1, outc, sem, seme, semr0, semr1):
        c = lax.axis_index("c")
        s = lax.axis_index("s")
        w = c * NS + s
        pltpu.sync_copy(zeros_hbm, acc_sh.at[pl.ds(s * RPT, RPT)])

        def zrow(e, carry):
            for j in range(4):
                outc[e, pl.ds(64 + j * 16, 16)] = jnp.zeros((16,), f32)
            return carry

        lax.fori_loop(0, CHUNK, zrow, 0)
        plsc.subcore_barrier()

        sidx = (sidx0, sidx1)
        rows = (rows0, rows1)
        semr = (semr0, semr1)

        def fire_head(ck, dx):
            pltpu.sync_copy(dst_hbm.at[pl.ds(ck * CHUNK, CHUNK)], dx)
            for h in range(HEADS):
                for j in range(CHUNK // 16):
                    sl = pl.ds(j * 16, 16)
                    didx[h, sl] = dx[sl] + (h * N_PAD)
            pltpu.async_copy(ex_hbm.at[pl.ds(HEADS * ck, HEADS)], exv, seme)
            for h in range(HEADS):
                pltpu.async_copy(rdt_hbm.at[didx.at[h]], rdv.at[h], sem)

        def chunk(ck, dx, dx_n, carry):
            base = ck * CHUNK
            pltpu.sync_copy(src_hbm.at[pl.ds(base, QE)], sidx0)
            pltpu.async_copy(xwg_hbm.at[sidx0], rows0, semr0)
            pltpu.make_async_copy(
                ex_hbm.at[pl.ds(HEADS * ck, HEADS)], exv, seme).wait()
            for h in range(HEADS):
                pltpu.make_async_copy(rdt_hbm.at[didx.at[h]], rdv.at[h], sem).wait()
            for h in range(HEADS):
                for j in range(CHUNK // 16):
                    sl = pl.ds(j * 16, 16)
                    alv[pl.ds(h * CHUNK + j * 16, 16)] = (
                        exv[h, sl] * rdv[h, sl] * 0.25
                    )
            fire_head(ck + 1, dx_n)
            for q in range(4):
                b = q % 2
                nb = 1 - b
                pltpu.make_async_copy(
                    xwg_hbm.at[sidx[b]], rows[b], semr[b]).wait()
                if q < 3:
                    pltpu.sync_copy(
                        src_hbm.at[pl.ds(base + (q + 1) * QE, QE)], sidx[nb])
                    pltpu.async_copy(xwg_hbm.at[sidx[nb]], rows[nb], semr[nb])

                def edge4(g, carry2, q=q, b=b):
                    go = g * 4
                    eo0 = go + q * QE
                    e16 = (eo0 // 16) * 16
                    lane0 = eo0 - e16
                    vv = [alv[pl.ds(h * CHUNK + e16, 16)] for h in range(HEADS)]
                    for sub in range(4):
                        lanev = jnp.full((16,), lane0 + sub, dtype=i32)
                        acc = [jnp.zeros((16,), f32) for _ in range(4)]
                        for h in range(HEADS):
                            ah = lax.gather(
                                vv[h], lanev[:, None],
                                lax.GatherDimensionNumbers(
                                    offset_dims=(), collapsed_slice_dims=(0,),
                                    start_index_map=(0,)),
                                (1,),
                                mode=lax.GatherScatterMode.PROMISE_IN_BOUNDS,
                            )
                            for j in range(4):
                                acc[j] = acc[j] + ah * rows[b][go + sub, pl.ds(h * 64 + j * 16, 16)]
                        for j in range(4):
                            outc[eo0 + sub, pl.ds(j * 16, 16)] = acc[j]
                    return carry2

                lax.fori_loop(0, QE // 4, edge4, 0)
            pltpu.sync_copy(outc, acc_sh.at[dx], add=True)
            return carry

        c0 = w * cpw
        fire_head(c0, dstidx)

        def body(k2, carry):
            ck = c0 + 2 * k2
            carry = chunk(ck, dstidx, dstidx2, carry)
            carry = chunk(ck + 1, dstidx2, dstidx, carry)
            return carry

        lax.fori_loop(0, cpw // 2, body, 0)
        pltpu.make_async_copy(
            ex_hbm.at[pl.ds(0, HEADS)], exv, seme).wait()
        for h in range(HEADS):
            pltpu.make_async_copy(rdt_hbm.at[didx.at[h]], rdv.at[h], sem).wait()
        plsc.subcore_barrier()
        pltpu.sync_copy(
            acc_sh.at[pl.ds(s * RPT, RPT)], out_hbm.at[c, pl.ds(s * RPT, RPT)]
        )

    return gat_b_kernel


def _row_spec(dd):
    return pl.BlockSpec((BLK, dd), lambda i: (i, 0))


def _rep_spec(r, dd):
    return pl.BlockSpec((r, dd), lambda i: (0, 0))


def _t0_body(x_ref, w_ref, d0_ref, d1_ref, dinv_ref, xs_ref):
    dinv = lax.rsqrt(d0_ref[...] + d1_ref[...])
    dinv_ref[...] = dinv
    xs_ref[...] = jnp.dot(x_ref[...], w_ref[...], preferred_element_type=f32) * dinv


def _t0(x, w0, d0, d1):
    return pl.pallas_call(
        _t0_body,
        grid=(N // BLK,),
        in_specs=[_row_spec(128), _rep_spec(128, 128), _row_spec(1), _row_spec(1)],
        out_specs=[_row_spec(1), _row_spec(128)],
        out_shape=[
            jax.ShapeDtypeStruct((N, 1), f32),
            jax.ShapeDtypeStruct((N, 128), f32),
        ],
    )(x, w0, d0, d1)


def _t1_body(p0, p1, dinv, b0, w1, h1_ref, xs_ref):
    dv = dinv[...]
    h1 = jnp.maximum((p0[...] + p1[...]) * dv + b0[...], 0.0)
    h1_ref[...] = h1
    xs_ref[...] = jnp.dot(h1, w1[...], preferred_element_type=f32) * dv


def _t1(p0, p1, dinv, b0, w1):
    return pl.pallas_call(
        _t1_body,
        grid=(N // BLK,),
        in_specs=[_row_spec(128), _row_spec(128), _row_spec(1),
                  _rep_spec(1, 128), _rep_spec(128, 128)],
        out_specs=[_row_spec(128), _row_spec(128)],
        out_shape=[
            jax.ShapeDtypeStruct((N, 128), f32),
            jax.ShapeDtypeStruct((N, 128), f32),
        ],
    )(p0, p1, dinv, b0, w1)


def _t2_body(p0, p1, dinv, b1, h1, w2, xs_ref):
    dv = dinv[...]
    h2 = h1[...] + jnp.maximum((p0[...] + p1[...]) * dv + b1[...], 0.0)
    xs_ref[...] = jnp.dot(h2, w2[...], preferred_element_type=f32) * dv


def _t2(p0, p1, dinv, b1, h1, w2):
    return pl.pallas_call(
        _t2_body,
        grid=(N // BLK,),
        in_specs=[_row_spec(128), _row_spec(128), _row_spec(1),
                  _rep_spec(1, 128), _row_spec(128), _rep_spec(128, 64)],
        out_specs=[_row_spec(64)],
        out_shape=[jax.ShapeDtypeStruct((N, 64), f32)],
    )(p0, p1, dinv, b1, h1, w2)[0]


def _t3_body(p0, p1, dinv, b2, wg, acomb, xwg_ref, aout_ref):
    dv = dinv[...]
    h3 = jnp.maximum((p0[...] + p1[...]) * dv + b2[...], 0.0)
    xwg = jnp.dot(h3, wg[...], preferred_element_type=f32)
    xwg_ref[...] = xwg
    aout_ref[...] = jnp.dot(xwg, acomb[...], preferred_element_type=f32)


def _t3(p0, p1, dinv, b2, wg, acomb):
    return pl.pallas_call(
        _t3_body,
        grid=(N // BLK,),
        in_specs=[_row_spec(64), _row_spec(64), _row_spec(1),
                  _rep_spec(1, 64), _rep_spec(64, 256), _rep_spec(256, 128)],
        out_specs=[_row_spec(256), _row_spec(128)],
        out_shape=[
            jax.ShapeDtypeStruct((N, 256), f32),
            jax.ShapeDtypeStruct((N, 128), f32),
        ],
    )(p0, p1, dinv, b2, wg, acomb)


def _t3b_body(asrc_ref, adst_ref, c_ref):
    amax = jnp.max(asrc_ref[...], axis=0, keepdims=True)
    x = amax + adst_ref[...]
    c_ref[...] = jnp.maximum(x, 0.0) + 0.2 * jnp.minimum(x, 0.0)


def _t3b(asrc, adst):
    return pl.pallas_call(
        _t3b_body,
        grid=(1,),
        in_specs=[_rep_spec(N, HEADS), _rep_spec(N, HEADS)],
        out_specs=[_rep_spec(N, HEADS)],
        out_shape=[jax.ShapeDtypeStruct((N, HEADS), f32)],
    )(asrc, adst)[0]


def _t4_body(d0_ref, d1_ref, r_ref):
    r_ref[...] = 1.0 / (d0_ref[...] + d1_ref[...] + 1e-16)


def _t4(d0, d1):
    r = HEADS * N_PAD // 128
    return pl.pallas_call(
        _t4_body,
        grid=(1,),
        in_specs=[_rep_spec(r, 128), _rep_spec(r, 128)],
        out_specs=[_rep_spec(r, 128)],
        out_shape=[jax.ShapeDtypeStruct((r, 128), f32)],
    )(d0, d1)[0]


def _t5_body(p0, p1, bg, wcomb, bcomb, h4_ref, ge_ref, bo_ref):
    h4 = p0[...] + p1[...] + bg[...]
    h4_ref[...] = h4

    @pl.when(pl.program_id(0) == 0)
    def _():
        ge_ref[...] = jnp.zeros_like(ge_ref)

    ge_ref[...] += jnp.sum(h4, axis=0, keepdims=True) * (1.0 / N)
    bo_ref[...] = jnp.dot(h4, wcomb[...], preferred_element_type=f32) + bcomb[...]


def _t5(p0, p1, bg, wcomb, bcomb):
    return pl.pallas_call(
        _t5_body,
        grid=(N // BLK,),
        in_specs=[_row_spec(64), _row_spec(64), _rep_spec(1, 64),
                  _rep_spec(64, 128), _rep_spec(1, 128)],
        out_specs=[_row_spec(64), _rep_spec(1, 64), _row_spec(128)],
        out_shape=[
            jax.ShapeDtypeStruct((N, 64), f32),
            jax.ShapeDtypeStruct((1, 64), f32),
            jax.ShapeDtypeStruct((N, 128), f32),
        ],
    )(p0, p1, bg, wcomb, bcomb)


def _pad_rows(t):
    return jnp.pad(t, ((0, N_PAD - N), (0, 0)))


def _flat_t(a):
    return jnp.pad(a.T, ((0, 0), (0, N_PAD - N))).reshape(-1)


def kernel(x, edge_index, W0, b0, W1, b1, W2, b2, Wg, att_src, att_dst, bg,
           Wb, bb, Wo, bo):
    e_in = edge_index.shape[1]
    etot = e_in + N
    cpw = -(-etot // (NW * CHUNK))
    cpw += cpw % 2
    epad = cpw * NW * CHUNK

    loop = jnp.arange(N, dtype=i32)
    fill = TRASH + (jnp.arange(epad - etot, dtype=i32) % (N_PAD - N))
    zc = jnp.zeros((CHUNK,), i32)
    srcp = jnp.concatenate([edge_index[0], loop, fill, zc])
    dstp = jnp.concatenate([edge_index[1], loop, fill, zc])
    nch = epad // CHUNK
    pack = jnp.stack([srcp[:epad].reshape(nch, CHUNK),
                      dstp[:epad].reshape(nch, CHUNK)],
                     axis=1).reshape(2 * nch, CHUNK)
    pack = jnp.concatenate([pack, jnp.zeros((2, CHUNK), i32)], axis=0)

    z1 = jnp.zeros((RPT,), f32)
    z128 = jnp.zeros((RPT, 128), f32)
    z64 = jnp.zeros((RPT, 64), f32)
    z4 = jnp.zeros((HEADS * RPT,), f32)

    deg = _make_deg(cpw)(dstp, z1)
    d0 = deg[0, :N].reshape(N, 1)
    d1 = deg[1, :N].reshape(N, 1)
    dinv, xw0s = _t0(x, W0, d0, d1)

    gcn128 = _make_gcn2(cpw)
    p = gcn128(pack, _pad_rows(xw0s), z128)
    h1, xw1s = _t1(p[0, :N], p[1, :N], dinv, b0.reshape(1, -1), W1)

    p = gcn128(pack, _pad_rows(xw1s), z128)
    xw2s = _t2(p[0, :N], p[1, :N], dinv, b1.reshape(1, -1), h1, W2)

    xw2s_wide = jnp.pad(xw2s, ((0, 0), (0, 64)))
    p = gcn128(pack, _pad_rows(xw2s_wide), z128)
    p = p[:, :, :64]

    acomb = jnp.zeros((256, 128), f32)
    for h in range(HEADS):
        acomb = acomb.at[h * 64:(h + 1) * 64, h].set(att_src[h])
        acomb = acomb.at[h * 64:(h + 1) * 64, HEADS + h].set(att_dst[h])
    xwg, aout = _t3(p[0, :N], p[1, :N], dinv, b2.reshape(1, -1), Wg, acomb)
    asrc = aout[:, :HEADS]
    adst = aout[:, HEADS:2 * HEADS]
    cvals = _t3b(asrc, adst)

    den, ex = _make_gat_a(epad, cpw)(
        srcp, dstp, _flat_t(asrc), _flat_t(adst), _flat_t(cvals), z4
    )
    ex = jnp.pad(ex, ((0, HEADS), (0, 0)))
    r = HEADS * N_PAD // 128
    rden = _t4(den[0].reshape(r, 128), den[1].reshape(r, 128)).reshape(-1)

    pg = _make_gat_b(epad, cpw)(srcp, dstp, ex, rden, _pad_rows(xwg), z128)

    wcomb = jnp.zeros((64, 128), f32)
    wcomb = wcomb.at[:, :2].set(Wb)
    wcomb = wcomb.at[:, 2:8].set(Wo)
    bcomb = jnp.zeros((1, 128), f32)
    bcomb = bcomb.at[0, :2].set(bb)
    bcomb = bcomb.at[0, 2:8].set(bo)

    h4, ge, bo_full = _t5(pg[0, :N, :64], pg[1, :N, :64], bg.reshape(1, -1), wcomb, bcomb)
    return h4, ge, bo_full[:, :2], bo_full[:, 2:8]
